# Initial kernel scaffold; baseline (speedup 1.0000x reference)
import functools

import jax
import jax.numpy as jnp
from jax import lax
from jax.experimental import pallas as pl
from jax.experimental.pallas import tpu as pltpu
from jax.experimental.pallas import tpu_sc as plsc

N_ENT = 10000
N_ITEMS = 5000
E = 320000
DIM = 128
N_REL = 11
DECAY = 1e-05
BATCH = 1024

NRM1 = N_REL - 1
ROWW = DIM + 16
NC, NS, L = 2, 16, 16
NW = NC * NS
EPW = E // NW
CH = 80
NCHUNK = EPW // CH
RPS = N_ENT // NS
ZR = 25
GPB = BATCH // NW

_mesh = plsc.VectorSubcoreMesh(core_axis_name="c", subcore_axis_name="s")

IB = 500
RB = 1000


def _attn_body(pic_ref, txt_ref, pfw_ref, pfb_ref, tfw_ref, tfb_ref, ww_ref,
               out_ref):
    ox = lax.dot_general(pic_ref[...], pfw_ref[...], (((1,), (1,)), ((), ())),
                         precision=lax.Precision.HIGHEST) + pfb_ref[...]
    oy = lax.dot_general(txt_ref[...], tfw_ref[...], (((1,), (1,)), ((), ())),
                         precision=lax.Precision.HIGHEST) + tfb_ref[...]
    a = jnp.maximum(jnp.sum(ox * ww_ref[...], axis=1, keepdims=True), 0.0)
    b = jnp.maximum(jnp.sum(oy * ww_ref[...], axis=1, keepdims=True), 0.0)
    ea = jnp.exp(a)
    eb = jnp.exp(b)
    out_ref[...] = (ea * ox + eb * oy) / (ea + eb)


def _attention(picture, text, PF_W, PF_b, TF_W, TF_b, w_W):
    return pl.pallas_call(
        _attn_body,
        grid=(N_ITEMS // IB,),
        in_specs=[
            pl.BlockSpec((IB, 2048), lambda i: (i, 0)),
            pl.BlockSpec((IB, 768), lambda i: (i, 0)),
            pl.BlockSpec((DIM, 2048), lambda i: (0, 0)),
            pl.BlockSpec((1, DIM), lambda i: (0, 0)),
            pl.BlockSpec((DIM, 768), lambda i: (0, 0)),
            pl.BlockSpec((1, DIM), lambda i: (0, 0)),
            pl.BlockSpec((1, DIM), lambda i: (0, 0)),
        ],
        out_specs=pl.BlockSpec((IB, DIM), lambda i: (i, 0)),
        out_shape=jax.ShapeDtypeStruct((N_ITEMS, DIM), jnp.float32),
    )(picture, text, PF_W, PF_b.reshape(1, DIM), TF_W, TF_b.reshape(1, DIM),
      w_W)


def _premult_body(ent_ref, w_ref, out_ref):
    ones = jnp.ones((RB, ROWW - DIM), jnp.float32)
    for r in range(NRM1):
        out_ref[r] = jnp.concatenate(
            [ent_ref[...] * w_ref[r:r + 1, :], ones], axis=1)


def _premult(ent, weight):
    out = pl.pallas_call(
        _premult_body,
        grid=(N_ENT // RB,),
        in_specs=[
            pl.BlockSpec((RB, DIM), lambda i: (i, 0)),
            pl.BlockSpec((NRM1, DIM), lambda i: (0, 0)),
        ],
        out_specs=pl.BlockSpec((NRM1, RB, ROWW), lambda i: (0, i, 0)),
        out_shape=jax.ShapeDtypeStruct((NRM1, N_ENT, ROWW), jnp.float32),
    )(ent, weight)
    return out.reshape(NRM1 * N_ENT, ROWW)


def _agg_normalize(part_ref):
    s = part_ref[0, :, :DIM] + part_ref[1, :, :DIM]
    c = part_ref[0, :, DIM:DIM + 1] + part_ref[1, :, DIM:DIM + 1]
    agg = s / jnp.maximum(c, 1.0)
    nrm = jnp.sqrt(jnp.sum(agg * agg, axis=1, keepdims=True))
    return agg / jnp.maximum(nrm, 1e-12)


def _norm_premult_body(part_ref, w_ref, ent_ref, entr_ref):
    ent = _agg_normalize(part_ref)
    ent_ref[...] = ent
    ones = jnp.ones((RB, ROWW - DIM), jnp.float32)
    for r in range(NRM1):
        entr_ref[r] = jnp.concatenate([ent * w_ref[r:r + 1, :], ones], axis=1)


def _norm_premult(part, weight):
    ent, entr = pl.pallas_call(
        _norm_premult_body,
        grid=(N_ENT // RB,),
        in_specs=[
            pl.BlockSpec((NC, RB, ROWW), lambda i: (0, i, 0)),
            pl.BlockSpec((NRM1, DIM), lambda i: (0, 0)),
        ],
        out_specs=[
            pl.BlockSpec((RB, DIM), lambda i: (i, 0)),
            pl.BlockSpec((NRM1, RB, ROWW), lambda i: (0, i, 0)),
        ],
        out_shape=[
            jax.ShapeDtypeStruct((N_ENT, DIM), jnp.float32),
            jax.ShapeDtypeStruct((NRM1, N_ENT, ROWW), jnp.float32),
        ],
    )(part, weight)
    return ent, entr.reshape(NRM1 * N_ENT, ROWW)


def _norm_final_body(part_ref, emb_ref, ent1_ref, out_ref):
    out_ref[...] = emb_ref[...] + ent1_ref[...] + _agg_normalize(part_ref)


def _norm_final(part, all_embed, ent1):
    return pl.pallas_call(
        _norm_final_body,
        grid=(N_ENT // RB,),
        in_specs=[
            pl.BlockSpec((NC, RB, ROWW), lambda i: (0, i, 0)),
            pl.BlockSpec((RB, DIM), lambda i: (i, 0)),
            pl.BlockSpec((RB, DIM), lambda i: (i, 0)),
        ],
        out_specs=pl.BlockSpec((RB, DIM), lambda i: (i, 0)),
        out_shape=jax.ShapeDtypeStruct((N_ENT, DIM), jnp.float32),
    )(part, all_embed, ent1)


def _sc_hop(entr, fidx, head):
    @functools.partial(
        pl.kernel,
        out_type=jax.ShapeDtypeStruct((NC * N_ENT, ROWW), jnp.float32),
        mesh=_mesh,
        scratch_types=[
            pltpu.VMEM((CH,), jnp.int32),
            pltpu.VMEM((CH,), jnp.int32),
            pltpu.VMEM((CH, ROWW), jnp.float32),
            pltpu.VMEM((ZR, ROWW), jnp.float32),
            pltpu.VMEM_SHARED((N_ENT, ROWW), jnp.float32),
            pltpu.SemaphoreType.DMA,
        ],
    )
    def k(entr_hbm, fidx_hbm, head_hbm, out_hbm, fidx_v, head_v, rows_v,
          zbuf, acc, sem):
        c = lax.axis_index("c")
        s = lax.axis_index("s")
        wid = c * NS + s

        @pl.loop(0, ZR)
        def _(i):
            @pl.loop(0, ROWW, step=L)
            def _(j):
                zbuf[i, pl.ds(j, L)] = jnp.zeros((L,), jnp.float32)

        @pl.loop(0, RPS // ZR)
        def _(k2):
            pltpu.sync_copy(zbuf, acc.at[pl.ds(s * RPS + k2 * ZR, ZR)])

        plsc.subcore_barrier()

        @pl.loop(0, NCHUNK)
        def _(k2):
            base = wid * EPW + k2 * CH
            pltpu.sync_copy(fidx_hbm.at[pl.ds(base, CH)], fidx_v)
            pltpu.sync_copy(head_hbm.at[pl.ds(base, CH)], head_v)
            pltpu.async_copy(entr_hbm.at[fidx_v], rows_v, sem).wait()
            pltpu.sync_copy(rows_v, acc.at[head_v], add=True)

        plsc.subcore_barrier()
        pltpu.sync_copy(acc.at[pl.ds(s * RPS, RPS)],
                        out_hbm.at[pl.ds(c * N_ENT + s * RPS, RPS)])

    return k(entr, fidx, head).reshape(NC, N_ENT, ROWW)


def _sc_gather(res2, item_emb, users, pos_items, neg_items):
    @functools.partial(
        pl.kernel,
        out_type=jax.ShapeDtypeStruct((3 * BATCH, DIM), jnp.float32),
        mesh=_mesh,
        scratch_types=[
            pltpu.VMEM((GPB,), jnp.int32),
            pltpu.VMEM((GPB, DIM), jnp.float32),
            pltpu.SemaphoreType.DMA,
        ],
    )
    def k(res_hbm, item_hbm, u_hbm, p_hbm, n_hbm, out_hbm, idx_v, buf, sem):
        c = lax.axis_index("c")
        s = lax.axis_index("s")
        base = (c * NS + s) * GPB
        pltpu.sync_copy(u_hbm.at[pl.ds(base, GPB)], idx_v)
        pltpu.async_copy(res_hbm.at[idx_v], buf, sem).wait()
        pltpu.sync_copy(buf, out_hbm.at[pl.ds(base, GPB)])
        pltpu.sync_copy(p_hbm.at[pl.ds(base, GPB)], idx_v)
        pltpu.async_copy(item_hbm.at[idx_v], buf, sem).wait()
        pltpu.sync_copy(buf, out_hbm.at[pl.ds(BATCH + base, GPB)])
        pltpu.sync_copy(n_hbm.at[pl.ds(base, GPB)], idx_v)
        pltpu.async_copy(item_hbm.at[idx_v], buf, sem).wait()
        pltpu.sync_copy(buf, out_hbm.at[pl.ds(2 * BATCH + base, GPB)])

    return k(res2, item_emb, users, pos_items, neg_items)


def _loss_body(g_ref, out_ref):
    g = g_ref[...]
    u = g[:BATCH]
    p = g[BATCH:2 * BATCH]
    n = g[2 * BATCH:]
    x = jnp.sum(u * p, axis=1, keepdims=True) - jnp.sum(
        u * n, axis=1, keepdims=True)
    ls = jnp.minimum(x, 0.0) - jnp.log1p(jnp.exp(-jnp.abs(x)))
    mf = -jnp.mean(ls)
    reg = 0.5 * (jnp.sum(u * u) + jnp.sum(p * p) + jnp.sum(n * n))
    emb = jnp.float32(DECAY / BATCH) * reg
    lane = lax.broadcasted_iota(jnp.int32, (1, DIM), 1)
    out_ref[...] = jnp.where(lane == 0, mf, jnp.where(lane == 1, emb, 0.0))


def _loss(gath):
    return pl.pallas_call(
        _loss_body,
        in_specs=[pl.BlockSpec((3 * BATCH, DIM), lambda: (0, 0))],
        out_specs=pl.BlockSpec((1, DIM), lambda: (0, 0)),
        out_shape=jax.ShapeDtypeStruct((1, DIM), jnp.float32),
    )(gath)


def kernel(picture, text, all_embed, weight, PF_W, PF_b, TF_W, TF_b, w_W,
           edge_index, edge_type, users, pos_items, neg_items):
    head = edge_index[0].astype(jnp.int32)
    tail = edge_index[1].astype(jnp.int32)
    fidx = (edge_type.astype(jnp.int32) - 1) * N_ENT + tail

    item_emb = _attention(picture, text, PF_W, PF_b, TF_W, TF_b, w_W)

    entr0 = _premult(all_embed, weight)
    part1 = _sc_hop(entr0, fidx, head)
    ent1, entr1 = _norm_premult(part1, weight)
    part2 = _sc_hop(entr1, fidx, head)
    res2 = _norm_final(part2, all_embed, ent1)

    gath = _sc_gather(res2, item_emb, users.astype(jnp.int32),
                      pos_items.astype(jnp.int32), neg_items.astype(jnp.int32))
    lossvec = _loss(gath)
    mf_loss = lossvec[0, 0]
    emb_loss = lossvec[0, 1]
    return (mf_loss + emb_loss, mf_loss, emb_loss)


# speedup vs baseline: 5.9489x; 5.9489x over previous
import functools

import jax
import jax.numpy as jnp
from jax import lax
from jax.experimental import pallas as pl
from jax.experimental.pallas import tpu as pltpu
from jax.experimental.pallas import tpu_sc as plsc

N_ENT = 10000
N_ITEMS = 5000
E = 320000
DIM = 128
N_REL = 11
DECAY = 1e-05
BATCH = 1024

NRM1 = N_REL - 1
NC, NS, L = 2, 16, 16
NW = NC * NS
EPW = E // NW
CH = 80
NCHUNK = EPW // CH
ARB = 80
NACH = N_ENT // ARB
GPB = BATCH // NW

IB = 1000
RB = 1000


def _mesh():
    return plsc.VectorSubcoreMesh(core_axis_name="c", subcore_axis_name="s",
                                  num_cores=NC, num_subcores=NS)


def _attn_body(pic_ref, txt_ref, pfw_ref, pfb_ref, tfw_ref, tfb_ref, ww_ref,
               out_ref):
    ox = lax.dot_general(pic_ref[...], pfw_ref[...], (((1,), (1,)), ((), ())),
                         precision=lax.Precision.HIGHEST) + pfb_ref[...]
    oy = lax.dot_general(txt_ref[...], tfw_ref[...], (((1,), (1,)), ((), ())),
                         precision=lax.Precision.HIGHEST) + tfb_ref[...]
    a = jnp.maximum(jnp.sum(ox * ww_ref[...], axis=1, keepdims=True), 0.0)
    b = jnp.maximum(jnp.sum(oy * ww_ref[...], axis=1, keepdims=True), 0.0)
    ea = jnp.exp(a)
    eb = jnp.exp(b)
    out_ref[...] = (ea * ox + eb * oy) / (ea + eb)


def _attention(picture, text, PF_W, PF_b, TF_W, TF_b, w_W):
    return pl.pallas_call(
        _attn_body,
        grid=(N_ITEMS // IB,),
        in_specs=[
            pl.BlockSpec((IB, 2048), lambda i: (i, 0)),
            pl.BlockSpec((IB, 768), lambda i: (i, 0)),
            pl.BlockSpec((DIM, 2048), lambda i: (0, 0)),
            pl.BlockSpec((1, DIM), lambda i: (0, 0)),
            pl.BlockSpec((DIM, 768), lambda i: (0, 0)),
            pl.BlockSpec((1, DIM), lambda i: (0, 0)),
            pl.BlockSpec((1, DIM), lambda i: (0, 0)),
        ],
        out_specs=pl.BlockSpec((IB, DIM), lambda i: (i, 0)),
        out_shape=jax.ShapeDtypeStruct((N_ITEMS, DIM), jnp.float32),
    )(picture, text, PF_W, PF_b.reshape(1, DIM), TF_W, TF_b.reshape(1, DIM),
      w_W)


def _premult_body(ent_ref, w_ref, out_ref):
    for r in range(NRM1):
        out_ref[r] = ent_ref[...] * w_ref[r:r + 1, :]


def _premult(ent, weight):
    out = pl.pallas_call(
        _premult_body,
        grid=(N_ENT // RB,),
        in_specs=[
            pl.BlockSpec((RB, DIM), lambda i: (i, 0)),
            pl.BlockSpec((NRM1, DIM), lambda i: (0, 0)),
        ],
        out_specs=pl.BlockSpec((NRM1, RB, DIM), lambda i: (0, i, 0)),
        out_shape=jax.ShapeDtypeStruct((NRM1, N_ENT, DIM), jnp.float32),
    )(ent, weight)
    return out.reshape(NRM1 * N_ENT, DIM)


def _sum_normalize(part_ref):
    s = part_ref[0] + part_ref[1]
    nrm = jnp.sqrt(jnp.sum(s * s, axis=1, keepdims=True))
    return s / jnp.maximum(nrm, 1e-12)


def _norm_premult_body(part_ref, w_ref, ent_ref, entr_ref):
    ent = _sum_normalize(part_ref)
    ent_ref[...] = ent
    for r in range(NRM1):
        entr_ref[r] = ent * w_ref[r:r + 1, :]


def _norm_premult(part, weight):
    ent, entr = pl.pallas_call(
        _norm_premult_body,
        grid=(N_ENT // RB,),
        in_specs=[
            pl.BlockSpec((NC, RB, DIM), lambda i: (0, i, 0)),
            pl.BlockSpec((NRM1, DIM), lambda i: (0, 0)),
        ],
        out_specs=[
            pl.BlockSpec((RB, DIM), lambda i: (i, 0)),
            pl.BlockSpec((NRM1, RB, DIM), lambda i: (0, i, 0)),
        ],
        out_shape=[
            jax.ShapeDtypeStruct((N_ENT, DIM), jnp.float32),
            jax.ShapeDtypeStruct((NRM1, N_ENT, DIM), jnp.float32),
        ],
    )(part, weight)
    return ent, entr.reshape(NRM1 * N_ENT, DIM)


def _norm_final_body(part_ref, emb_ref, ent1_ref, out_ref):
    out_ref[...] = emb_ref[...] + ent1_ref[...] + _sum_normalize(part_ref)


def _norm_final(part, all_embed, ent1):
    return pl.pallas_call(
        _norm_final_body,
        grid=(N_ENT // RB,),
        in_specs=[
            pl.BlockSpec((NC, RB, DIM), lambda i: (0, i, 0)),
            pl.BlockSpec((RB, DIM), lambda i: (i, 0)),
            pl.BlockSpec((RB, DIM), lambda i: (i, 0)),
        ],
        out_specs=pl.BlockSpec((RB, DIM), lambda i: (i, 0)),
        out_shape=jax.ShapeDtypeStruct((N_ENT, DIM), jnp.float32),
    )(part, all_embed, ent1)


def _sc_hop(entr, fidx, head):
    @functools.partial(
        pl.kernel,
        out_type=jax.ShapeDtypeStruct((NC * N_ENT, DIM), jnp.float32),
        mesh=_mesh(),
        scratch_types=[
            pltpu.VMEM((CH,), jnp.int32),
            pltpu.VMEM((CH,), jnp.int32),
            pltpu.VMEM((CH, DIM), jnp.float32),
            pltpu.VMEM((ARB, DIM), jnp.float32),
            pltpu.VMEM_SHARED((N_ENT, DIM), jnp.float32),
            pltpu.SemaphoreType.DMA,
        ],
    )
    def k(entr_hbm, fidx_hbm, head_hbm, out_hbm, fidx_v, head_v, rows_v,
          zbuf, acc, sem):
        c = lax.axis_index("c")
        s = lax.axis_index("s")
        wid = c * NS + s

        @pl.loop(0, ARB)
        def _(i):
            @pl.loop(0, DIM, step=L)
            def _(j):
                zbuf[i, pl.ds(j, L)] = jnp.zeros((L,), jnp.float32)

        @pl.loop(s, NACH, step=NS)
        def _(g):
            pltpu.sync_copy(zbuf, acc.at[pl.ds(g * ARB, ARB)])

        plsc.subcore_barrier()

        @pl.loop(0, NCHUNK)
        def _(k2):
            base = wid * EPW + k2 * CH
            pltpu.sync_copy(fidx_hbm.at[pl.ds(base, CH)], fidx_v)
            pltpu.sync_copy(head_hbm.at[pl.ds(base, CH)], head_v)
            pltpu.async_copy(entr_hbm.at[fidx_v], rows_v, sem).wait()
            pltpu.sync_copy(rows_v, acc.at[head_v], add=True)

        plsc.subcore_barrier()

        @pl.loop(s, NACH, step=NS)
        def _(g):
            pltpu.sync_copy(acc.at[pl.ds(g * ARB, ARB)],
                            out_hbm.at[pl.ds(c * N_ENT + g * ARB, ARB)])

    return k(entr, fidx, head).reshape(NC, N_ENT, DIM)


def _sc_gather(res2, item_emb, users, pos_items, neg_items):
    @functools.partial(
        pl.kernel,
        out_type=jax.ShapeDtypeStruct((3 * BATCH, DIM), jnp.float32),
        mesh=_mesh(),
        scratch_types=[
            pltpu.VMEM((GPB,), jnp.int32),
            pltpu.VMEM((GPB, DIM), jnp.float32),
            pltpu.SemaphoreType.DMA,
        ],
    )
    def k(res_hbm, item_hbm, u_hbm, p_hbm, n_hbm, out_hbm, idx_v, buf, sem):
        c = lax.axis_index("c")
        s = lax.axis_index("s")
        base = (c * NS + s) * GPB
        pltpu.sync_copy(u_hbm.at[pl.ds(base, GPB)], idx_v)
        pltpu.async_copy(res_hbm.at[idx_v], buf, sem).wait()
        pltpu.sync_copy(buf, out_hbm.at[pl.ds(base, GPB)])
        pltpu.sync_copy(p_hbm.at[pl.ds(base, GPB)], idx_v)
        pltpu.async_copy(item_hbm.at[idx_v], buf, sem).wait()
        pltpu.sync_copy(buf, out_hbm.at[pl.ds(BATCH + base, GPB)])
        pltpu.sync_copy(n_hbm.at[pl.ds(base, GPB)], idx_v)
        pltpu.async_copy(item_hbm.at[idx_v], buf, sem).wait()
        pltpu.sync_copy(buf, out_hbm.at[pl.ds(2 * BATCH + base, GPB)])

    return k(res2, item_emb, users, pos_items, neg_items)


def _loss_body(g_ref, out_ref):
    g = g_ref[...]
    u = g[:BATCH]
    p = g[BATCH:2 * BATCH]
    n = g[2 * BATCH:]
    x = jnp.sum(u * p, axis=1, keepdims=True) - jnp.sum(
        u * n, axis=1, keepdims=True)
    ls = jnp.minimum(x, 0.0) - jnp.log1p(jnp.exp(-jnp.abs(x)))
    mf = -jnp.mean(ls)
    reg = 0.5 * (jnp.sum(u * u) + jnp.sum(p * p) + jnp.sum(n * n))
    emb = jnp.float32(DECAY / BATCH) * reg
    lane = lax.broadcasted_iota(jnp.int32, (1, DIM), 1)
    out_ref[...] = jnp.where(lane == 0, mf, jnp.where(lane == 1, emb, 0.0))


def _loss(gath):
    return pl.pallas_call(
        _loss_body,
        in_specs=[pl.BlockSpec((3 * BATCH, DIM), lambda: (0, 0))],
        out_specs=pl.BlockSpec((1, DIM), lambda: (0, 0)),
        out_shape=jax.ShapeDtypeStruct((1, DIM), jnp.float32),
    )(gath)


def kernel(picture, text, all_embed, weight, PF_W, PF_b, TF_W, TF_b, w_W,
           edge_index, edge_type, users, pos_items, neg_items):
    head = edge_index[0].astype(jnp.int32)
    tail = edge_index[1].astype(jnp.int32)
    fidx = (edge_type.astype(jnp.int32) - 1) * N_ENT + tail

    item_emb = _attention(picture, text, PF_W, PF_b, TF_W, TF_b, w_W)

    entr0 = _premult(all_embed, weight)
    part1 = _sc_hop(entr0, fidx, head)
    ent1, entr1 = _norm_premult(part1, weight)
    part2 = _sc_hop(entr1, fidx, head)
    res2 = _norm_final(part2, all_embed, ent1)

    gath = _sc_gather(res2, item_emb, users.astype(jnp.int32),
                      pos_items.astype(jnp.int32), neg_items.astype(jnp.int32))
    lossvec = _loss(gath)
    mf_loss = lossvec[0, 0]
    emb_loss = lossvec[0, 1]
    return (mf_loss + emb_loss, mf_loss, emb_loss)


# speedup vs baseline: 11.2575x; 1.8924x over previous
import functools

import jax
import jax.numpy as jnp
from jax import lax
from jax.experimental import pallas as pl
from jax.experimental.pallas import tpu as pltpu
from jax.experimental.pallas import tpu_sc as plsc

N_ENT = 10000
N_ITEMS = 5000
E = 320000
DIM = 128
N_REL = 11
DECAY = 1e-05
BATCH = 1024

NRM1 = N_REL - 1
NC, NS, L = 2, 16, 16
NW = NC * NS
EPW = E // NW
CH = 80
NCHUNK = EPW // CH
ARB = 80
NACH = N_ENT // ARB
GPB = BATCH // NW

IB = 1000
RB = 1000


def _mesh():
    return plsc.VectorSubcoreMesh(core_axis_name="c", subcore_axis_name="s",
                                  num_cores=NC, num_subcores=NS)


def _attn_body(pic_ref, txt_ref, pfw_ref, pfb_ref, tfw_ref, tfb_ref, ww_ref,
               out_ref):
    ox = lax.dot_general(pic_ref[...], pfw_ref[...], (((1,), (1,)), ((), ())),
                         precision=lax.Precision.HIGHEST) + pfb_ref[...]
    oy = lax.dot_general(txt_ref[...], tfw_ref[...], (((1,), (1,)), ((), ())),
                         precision=lax.Precision.HIGHEST) + tfb_ref[...]
    a = jnp.maximum(jnp.sum(ox * ww_ref[...], axis=1, keepdims=True), 0.0)
    b = jnp.maximum(jnp.sum(oy * ww_ref[...], axis=1, keepdims=True), 0.0)
    ea = jnp.exp(a)
    eb = jnp.exp(b)
    out_ref[...] = (ea * ox + eb * oy) / (ea + eb)


def _attention(picture, text, PF_W, PF_b, TF_W, TF_b, w_W):
    return pl.pallas_call(
        _attn_body,
        grid=(N_ITEMS // IB,),
        in_specs=[
            pl.BlockSpec((IB, 2048), lambda i: (i, 0)),
            pl.BlockSpec((IB, 768), lambda i: (i, 0)),
            pl.BlockSpec((DIM, 2048), lambda i: (0, 0)),
            pl.BlockSpec((1, DIM), lambda i: (0, 0)),
            pl.BlockSpec((DIM, 768), lambda i: (0, 0)),
            pl.BlockSpec((1, DIM), lambda i: (0, 0)),
            pl.BlockSpec((1, DIM), lambda i: (0, 0)),
        ],
        out_specs=pl.BlockSpec((IB, DIM), lambda i: (i, 0)),
        out_shape=jax.ShapeDtypeStruct((N_ITEMS, DIM), jnp.float32),
    )(picture, text, PF_W, PF_b.reshape(1, DIM), TF_W, TF_b.reshape(1, DIM),
      w_W)


def _premult_body(ent_ref, w_ref, out_ref):
    for r in range(NRM1):
        out_ref[r] = ent_ref[...] * w_ref[r:r + 1, :]


def _premult(ent, weight):
    out = pl.pallas_call(
        _premult_body,
        grid=(N_ENT // RB,),
        in_specs=[
            pl.BlockSpec((RB, DIM), lambda i: (i, 0)),
            pl.BlockSpec((NRM1, DIM), lambda i: (0, 0)),
        ],
        out_specs=pl.BlockSpec((NRM1, RB, DIM), lambda i: (0, i, 0)),
        out_shape=jax.ShapeDtypeStruct((NRM1, N_ENT, DIM), jnp.float32),
    )(ent, weight)
    return out.reshape(NRM1 * N_ENT, DIM)


def _sum_normalize(part_ref):
    s = part_ref[0] + part_ref[1]
    nrm = jnp.sqrt(jnp.sum(s * s, axis=1, keepdims=True))
    return s / jnp.maximum(nrm, 1e-12)


def _norm_premult_body(part_ref, w_ref, ent_ref, entr_ref):
    ent = _sum_normalize(part_ref)
    ent_ref[...] = ent
    for r in range(NRM1):
        entr_ref[r] = ent * w_ref[r:r + 1, :]


def _norm_premult(part, weight):
    ent, entr = pl.pallas_call(
        _norm_premult_body,
        grid=(N_ENT // RB,),
        in_specs=[
            pl.BlockSpec((NC, RB, DIM), lambda i: (0, i, 0)),
            pl.BlockSpec((NRM1, DIM), lambda i: (0, 0)),
        ],
        out_specs=[
            pl.BlockSpec((RB, DIM), lambda i: (i, 0)),
            pl.BlockSpec((NRM1, RB, DIM), lambda i: (0, i, 0)),
        ],
        out_shape=[
            jax.ShapeDtypeStruct((N_ENT, DIM), jnp.float32),
            jax.ShapeDtypeStruct((NRM1, N_ENT, DIM), jnp.float32),
        ],
    )(part, weight)
    return ent, entr.reshape(NRM1 * N_ENT, DIM)


def _norm_final_body(part_ref, emb_ref, ent1_ref, out_ref):
    out_ref[...] = emb_ref[...] + ent1_ref[...] + _sum_normalize(part_ref)


def _norm_final(part, all_embed, ent1):
    return pl.pallas_call(
        _norm_final_body,
        grid=(N_ENT // RB,),
        in_specs=[
            pl.BlockSpec((NC, RB, DIM), lambda i: (0, i, 0)),
            pl.BlockSpec((RB, DIM), lambda i: (i, 0)),
            pl.BlockSpec((RB, DIM), lambda i: (i, 0)),
        ],
        out_specs=pl.BlockSpec((RB, DIM), lambda i: (i, 0)),
        out_shape=jax.ShapeDtypeStruct((N_ENT, DIM), jnp.float32),
    )(part, all_embed, ent1)


NB = 4
NROUND = (NCHUNK - 1) // NB


def _sc_hop(entr, fidx, head):
    idx_scratch = [pltpu.VMEM((CH,), jnp.int32) for _ in range(2 * NB)]

    @functools.partial(
        pl.kernel,
        out_type=jax.ShapeDtypeStruct((NC * N_ENT, DIM), jnp.float32),
        mesh=_mesh(),
        scratch_types=idx_scratch + [
            pltpu.VMEM((NB, CH, DIM), jnp.float32),
            pltpu.VMEM_SHARED((N_ENT, DIM), jnp.float32),
            pltpu.SemaphoreType.DMA((NB,)),
            pltpu.SemaphoreType.DMA((NB,)),
            pltpu.SemaphoreType.DMA((NB,)),
        ],
    )
    def k(entr_hbm, fidx_hbm, head_hbm, out_hbm, *refs):
        fidx_v = refs[:NB]
        head_v = refs[NB:2 * NB]
        rows_v, acc, isem, gsem, ssem = refs[2 * NB:]
        c = lax.axis_index("c")
        s = lax.axis_index("s")
        wid = c * NS + s
        zbuf = rows_v.at[0]

        @pl.loop(0, ARB)
        def _(i):
            @pl.loop(0, DIM, step=L)
            def _(j):
                zbuf[i, pl.ds(j, L)] = jnp.zeros((L,), jnp.float32)

        @pl.loop(s, NACH, step=NS)
        def _(g):
            pltpu.sync_copy(zbuf, acc.at[pl.ds(g * ARB, ARB)])

        for b in range(1, NB):
            base = wid * EPW + (1 + b) * CH
            pltpu.async_copy(fidx_hbm.at[pl.ds(base, CH)], fidx_v[b],
                             isem.at[b])
            pltpu.async_copy(head_hbm.at[pl.ds(base, CH)], head_v[b],
                             isem.at[b])

        plsc.subcore_barrier()

        base0 = wid * EPW
        pltpu.sync_copy(fidx_hbm.at[pl.ds(base0, CH)], fidx_v[0])
        pltpu.sync_copy(head_hbm.at[pl.ds(base0, CH)], head_v[0])
        pltpu.async_copy(entr_hbm.at[fidx_v[0]], rows_v.at[0],
                         gsem.at[0]).wait()
        pltpu.sync_copy(rows_v.at[0], acc.at[head_v[0]], add=True)
        pltpu.async_copy(fidx_hbm.at[pl.ds(base0 + CH, CH)], fidx_v[0],
                         isem.at[0])
        pltpu.async_copy(head_hbm.at[pl.ds(base0 + CH, CH)], head_v[0],
                         isem.at[0])

        @pl.loop(0, NROUND)
        def _(m):
            for b in range(NB):
                base = wid * EPW + (1 + m * NB + b) * CH
                pltpu.make_async_copy(fidx_hbm.at[pl.ds(base, CH)],
                                      fidx_v[b], isem.at[b]).wait()
                pltpu.make_async_copy(head_hbm.at[pl.ds(base, CH)],
                                      head_v[b], isem.at[b]).wait()
                pltpu.async_copy(entr_hbm.at[fidx_v[b]], rows_v.at[b],
                                 gsem.at[b])
            for b in range(NB):
                pltpu.make_async_copy(entr_hbm.at[fidx_v[b]], rows_v.at[b],
                                      gsem.at[b]).wait()
                pltpu.async_copy(rows_v.at[b], acc.at[head_v[b]], ssem.at[b],
                                 add=True)
                pltpu.make_async_copy(rows_v.at[b], acc.at[head_v[b]],
                                      ssem.at[b]).wait()

                @pl.when(m < NROUND - 1)
                def _():
                    nbase = wid * EPW + (1 + (m + 1) * NB + b) * CH
                    pltpu.async_copy(fidx_hbm.at[pl.ds(nbase, CH)],
                                     fidx_v[b], isem.at[b])
                    pltpu.async_copy(head_hbm.at[pl.ds(nbase, CH)],
                                     head_v[b], isem.at[b])

        plsc.subcore_barrier()

        @pl.loop(s, NACH, step=NS)
        def _(g):
            pltpu.sync_copy(acc.at[pl.ds(g * ARB, ARB)],
                            out_hbm.at[pl.ds(c * N_ENT + g * ARB, ARB)])

    return k(entr, fidx, head).reshape(NC, N_ENT, DIM)


def _sc_gather(res2, item_emb, users, pos_items, neg_items):
    @functools.partial(
        pl.kernel,
        out_type=jax.ShapeDtypeStruct((3 * BATCH, DIM), jnp.float32),
        mesh=_mesh(),
        scratch_types=[
            pltpu.VMEM((GPB,), jnp.int32),
            pltpu.VMEM((GPB, DIM), jnp.float32),
            pltpu.SemaphoreType.DMA,
        ],
    )
    def k(res_hbm, item_hbm, u_hbm, p_hbm, n_hbm, out_hbm, idx_v, buf, sem):
        c = lax.axis_index("c")
        s = lax.axis_index("s")
        base = (c * NS + s) * GPB
        pltpu.sync_copy(u_hbm.at[pl.ds(base, GPB)], idx_v)
        pltpu.async_copy(res_hbm.at[idx_v], buf, sem).wait()
        pltpu.sync_copy(buf, out_hbm.at[pl.ds(base, GPB)])
        pltpu.sync_copy(p_hbm.at[pl.ds(base, GPB)], idx_v)
        pltpu.async_copy(item_hbm.at[idx_v], buf, sem).wait()
        pltpu.sync_copy(buf, out_hbm.at[pl.ds(BATCH + base, GPB)])
        pltpu.sync_copy(n_hbm.at[pl.ds(base, GPB)], idx_v)
        pltpu.async_copy(item_hbm.at[idx_v], buf, sem).wait()
        pltpu.sync_copy(buf, out_hbm.at[pl.ds(2 * BATCH + base, GPB)])

    return k(res2, item_emb, users, pos_items, neg_items)


def _loss_body(g_ref, out_ref):
    g = g_ref[...]
    u = g[:BATCH]
    p = g[BATCH:2 * BATCH]
    n = g[2 * BATCH:]
    x = jnp.sum(u * p, axis=1, keepdims=True) - jnp.sum(
        u * n, axis=1, keepdims=True)
    ls = jnp.minimum(x, 0.0) - jnp.log1p(jnp.exp(-jnp.abs(x)))
    mf = -jnp.mean(ls)
    reg = 0.5 * (jnp.sum(u * u) + jnp.sum(p * p) + jnp.sum(n * n))
    emb = jnp.float32(DECAY / BATCH) * reg
    lane = lax.broadcasted_iota(jnp.int32, (1, DIM), 1)
    out_ref[...] = jnp.where(lane == 0, mf, jnp.where(lane == 1, emb, 0.0))


def _loss(gath):
    return pl.pallas_call(
        _loss_body,
        in_specs=[pl.BlockSpec((3 * BATCH, DIM), lambda: (0, 0))],
        out_specs=pl.BlockSpec((1, DIM), lambda: (0, 0)),
        out_shape=jax.ShapeDtypeStruct((1, DIM), jnp.float32),
    )(gath)


def kernel(picture, text, all_embed, weight, PF_W, PF_b, TF_W, TF_b, w_W,
           edge_index, edge_type, users, pos_items, neg_items):
    head = edge_index[0].astype(jnp.int32)
    tail = edge_index[1].astype(jnp.int32)
    fidx = (edge_type.astype(jnp.int32) - 1) * N_ENT + tail

    item_emb = _attention(picture, text, PF_W, PF_b, TF_W, TF_b, w_W)

    entr0 = _premult(all_embed, weight)
    part1 = _sc_hop(entr0, fidx, head)
    ent1, entr1 = _norm_premult(part1, weight)
    part2 = _sc_hop(entr1, fidx, head)
    res2 = _norm_final(part2, all_embed, ent1)

    gath = _sc_gather(res2, item_emb, users.astype(jnp.int32),
                      pos_items.astype(jnp.int32), neg_items.astype(jnp.int32))
    lossvec = _loss(gath)
    mf_loss = lossvec[0, 0]
    emb_loss = lossvec[0, 1]
    return (mf_loss + emb_loss, mf_loss, emb_loss)


# speedup vs baseline: 14.3965x; 1.2788x over previous
import functools

import jax
import jax.numpy as jnp
from jax import lax
from jax.experimental import pallas as pl
from jax.experimental.pallas import tpu as pltpu
from jax.experimental.pallas import tpu_sc as plsc

N_ENT = 10000
N_ITEMS = 5000
E = 320000
DIM = 128
N_REL = 11
DECAY = 1e-05
BATCH = 1024

NRM1 = N_REL - 1
NC, NS, L = 2, 16, 16
NW = NC * NS
EPW = E // NW
CH = 40
NCHUNK = EPW // CH
ARB = CH
NACH = N_ENT // ARB
GPB = BATCH // NW

IB = 1000
RB = 1000


def _mesh():
    return plsc.VectorSubcoreMesh(core_axis_name="c", subcore_axis_name="s",
                                  num_cores=NC, num_subcores=NS)


def _attn_body(pic_ref, txt_ref, pfw_ref, pfb_ref, tfw_ref, tfb_ref, ww_ref,
               out_ref):
    ox = lax.dot_general(pic_ref[...], pfw_ref[...], (((1,), (1,)), ((), ())),
                         precision=lax.Precision.HIGHEST) + pfb_ref[...]
    oy = lax.dot_general(txt_ref[...], tfw_ref[...], (((1,), (1,)), ((), ())),
                         precision=lax.Precision.HIGHEST) + tfb_ref[...]
    a = jnp.maximum(jnp.sum(ox * ww_ref[...], axis=1, keepdims=True), 0.0)
    b = jnp.maximum(jnp.sum(oy * ww_ref[...], axis=1, keepdims=True), 0.0)
    ea = jnp.exp(a)
    eb = jnp.exp(b)
    out_ref[...] = (ea * ox + eb * oy) / (ea + eb)


def _attention(picture, text, PF_W, PF_b, TF_W, TF_b, w_W):
    return pl.pallas_call(
        _attn_body,
        grid=(N_ITEMS // IB,),
        in_specs=[
            pl.BlockSpec((IB, 2048), lambda i: (i, 0)),
            pl.BlockSpec((IB, 768), lambda i: (i, 0)),
            pl.BlockSpec((DIM, 2048), lambda i: (0, 0)),
            pl.BlockSpec((1, DIM), lambda i: (0, 0)),
            pl.BlockSpec((DIM, 768), lambda i: (0, 0)),
            pl.BlockSpec((1, DIM), lambda i: (0, 0)),
            pl.BlockSpec((1, DIM), lambda i: (0, 0)),
        ],
        out_specs=pl.BlockSpec((IB, DIM), lambda i: (i, 0)),
        out_shape=jax.ShapeDtypeStruct((N_ITEMS, DIM), jnp.float32),
    )(picture, text, PF_W, PF_b.reshape(1, DIM), TF_W, TF_b.reshape(1, DIM),
      w_W)


def _premult_body(ent_ref, w_ref, out_ref):
    for r in range(NRM1):
        out_ref[r] = ent_ref[...] * w_ref[r:r + 1, :]


def _premult(ent, weight):
    out = pl.pallas_call(
        _premult_body,
        grid=(N_ENT // RB,),
        in_specs=[
            pl.BlockSpec((RB, DIM), lambda i: (i, 0)),
            pl.BlockSpec((NRM1, DIM), lambda i: (0, 0)),
        ],
        out_specs=pl.BlockSpec((NRM1, RB, DIM), lambda i: (0, i, 0)),
        out_shape=jax.ShapeDtypeStruct((NRM1, N_ENT, DIM), jnp.float32),
    )(ent, weight)
    return out.reshape(NRM1 * N_ENT, DIM)


def _sum_normalize(part_ref):
    s = part_ref[0] + part_ref[1]
    nrm = jnp.sqrt(jnp.sum(s * s, axis=1, keepdims=True))
    return s / jnp.maximum(nrm, 1e-12)


def _norm_premult_body(part_ref, w_ref, ent_ref, entr_ref):
    ent = _sum_normalize(part_ref)
    ent_ref[...] = ent
    for r in range(NRM1):
        entr_ref[r] = ent * w_ref[r:r + 1, :]


def _norm_premult(part, weight):
    ent, entr = pl.pallas_call(
        _norm_premult_body,
        grid=(N_ENT // RB,),
        in_specs=[
            pl.BlockSpec((NC, RB, DIM), lambda i: (0, i, 0)),
            pl.BlockSpec((NRM1, DIM), lambda i: (0, 0)),
        ],
        out_specs=[
            pl.BlockSpec((RB, DIM), lambda i: (i, 0)),
            pl.BlockSpec((NRM1, RB, DIM), lambda i: (0, i, 0)),
        ],
        out_shape=[
            jax.ShapeDtypeStruct((N_ENT, DIM), jnp.float32),
            jax.ShapeDtypeStruct((NRM1, N_ENT, DIM), jnp.float32),
        ],
    )(part, weight)
    return ent, entr.reshape(NRM1 * N_ENT, DIM)


def _norm_final_body(part_ref, emb_ref, ent1_ref, out_ref):
    out_ref[...] = emb_ref[...] + ent1_ref[...] + _sum_normalize(part_ref)


def _norm_final(part, all_embed, ent1):
    return pl.pallas_call(
        _norm_final_body,
        grid=(N_ENT // RB,),
        in_specs=[
            pl.BlockSpec((NC, RB, DIM), lambda i: (0, i, 0)),
            pl.BlockSpec((RB, DIM), lambda i: (i, 0)),
            pl.BlockSpec((RB, DIM), lambda i: (i, 0)),
        ],
        out_specs=pl.BlockSpec((RB, DIM), lambda i: (i, 0)),
        out_shape=jax.ShapeDtypeStruct((N_ENT, DIM), jnp.float32),
    )(part, all_embed, ent1)


NB = 5
NROUND = NCHUNK // NB


def _sc_hop(entr, fidx, head):
    @functools.partial(
        pl.kernel,
        out_type=jax.ShapeDtypeStruct((NC * N_ENT, DIM), jnp.float32),
        mesh=_mesh(),
        scratch_types=[
            pltpu.VMEM((EPW,), jnp.int32),
            pltpu.VMEM((EPW,), jnp.int32),
            pltpu.VMEM((NB, CH, DIM), jnp.float32),
            pltpu.VMEM_SHARED((N_ENT, DIM), jnp.float32),
            pltpu.SemaphoreType.DMA((NB,)),
            pltpu.SemaphoreType.DMA((NB,)),
        ],
    )
    def k(entr_hbm, fidx_hbm, head_hbm, out_hbm, fidx_t, head_t, rows_v,
          acc, gsem, ssem):
        c = lax.axis_index("c")
        s = lax.axis_index("s")
        wid = c * NS + s
        zbuf = rows_v.at[0]

        pltpu.sync_copy(fidx_hbm.at[pl.ds(wid * EPW, EPW)], fidx_t)
        pltpu.sync_copy(head_hbm.at[pl.ds(wid * EPW, EPW)], head_t)

        @pl.loop(0, ARB)
        def _(i):
            @pl.loop(0, DIM, step=L)
            def _(j):
                zbuf[i, pl.ds(j, L)] = jnp.zeros((L,), jnp.float32)

        @pl.loop(s, NACH, step=NS)
        def _(g):
            pltpu.sync_copy(zbuf, acc.at[pl.ds(g * ARB, ARB)])

        plsc.subcore_barrier()

        for b in range(NB):
            pltpu.async_copy(entr_hbm.at[fidx_t.at[pl.ds(b * CH, CH)]],
                             rows_v.at[b], gsem.at[b])

        @pl.loop(0, NROUND)
        def _(m):
            for b in range(NB):
                q = m * NB + b
                pltpu.make_async_copy(
                    entr_hbm.at[fidx_t.at[pl.ds(q * CH, CH)]],
                    rows_v.at[b], gsem.at[b]).wait()
                pltpu.async_copy(rows_v.at[b],
                                 acc.at[head_t.at[pl.ds(q * CH, CH)]],
                                 ssem.at[b], add=True)

                @pl.when(m < NROUND - 1)
                def _():
                    pltpu.make_async_copy(
                        rows_v.at[b],
                        acc.at[head_t.at[pl.ds(q * CH, CH)]],
                        ssem.at[b]).wait()
                    pltpu.async_copy(
                        entr_hbm.at[fidx_t.at[pl.ds((q + NB) * CH, CH)]],
                        rows_v.at[b], gsem.at[b])

        for b in range(NB):
            q = NCHUNK - NB + b
            pltpu.make_async_copy(rows_v.at[b],
                                  acc.at[head_t.at[pl.ds(q * CH, CH)]],
                                  ssem.at[b]).wait()

        plsc.subcore_barrier()

        @pl.loop(s, NACH, step=NS)
        def _(g):
            pltpu.sync_copy(acc.at[pl.ds(g * ARB, ARB)],
                            out_hbm.at[pl.ds(c * N_ENT + g * ARB, ARB)])

    return k(entr, fidx, head).reshape(NC, N_ENT, DIM)


def _sc_gather(res2, item_emb, users, pos_items, neg_items):
    @functools.partial(
        pl.kernel,
        out_type=jax.ShapeDtypeStruct((3 * BATCH, DIM), jnp.float32),
        mesh=_mesh(),
        scratch_types=[
            pltpu.VMEM((GPB,), jnp.int32),
            pltpu.VMEM((GPB, DIM), jnp.float32),
            pltpu.SemaphoreType.DMA,
        ],
    )
    def k(res_hbm, item_hbm, u_hbm, p_hbm, n_hbm, out_hbm, idx_v, buf, sem):
        c = lax.axis_index("c")
        s = lax.axis_index("s")
        base = (c * NS + s) * GPB
        pltpu.sync_copy(u_hbm.at[pl.ds(base, GPB)], idx_v)
        pltpu.async_copy(res_hbm.at[idx_v], buf, sem).wait()
        pltpu.sync_copy(buf, out_hbm.at[pl.ds(base, GPB)])
        pltpu.sync_copy(p_hbm.at[pl.ds(base, GPB)], idx_v)
        pltpu.async_copy(item_hbm.at[idx_v], buf, sem).wait()
        pltpu.sync_copy(buf, out_hbm.at[pl.ds(BATCH + base, GPB)])
        pltpu.sync_copy(n_hbm.at[pl.ds(base, GPB)], idx_v)
        pltpu.async_copy(item_hbm.at[idx_v], buf, sem).wait()
        pltpu.sync_copy(buf, out_hbm.at[pl.ds(2 * BATCH + base, GPB)])

    return k(res2, item_emb, users, pos_items, neg_items)


def _loss_body(g_ref, out_ref):
    g = g_ref[...]
    u = g[:BATCH]
    p = g[BATCH:2 * BATCH]
    n = g[2 * BATCH:]
    x = jnp.sum(u * p, axis=1, keepdims=True) - jnp.sum(
        u * n, axis=1, keepdims=True)
    ls = jnp.minimum(x, 0.0) - jnp.log1p(jnp.exp(-jnp.abs(x)))
    mf = -jnp.mean(ls)
    reg = 0.5 * (jnp.sum(u * u) + jnp.sum(p * p) + jnp.sum(n * n))
    emb = jnp.float32(DECAY / BATCH) * reg
    lane = lax.broadcasted_iota(jnp.int32, (1, DIM), 1)
    out_ref[...] = jnp.where(lane == 0, mf, jnp.where(lane == 1, emb, 0.0))


def _loss(gath):
    return pl.pallas_call(
        _loss_body,
        in_specs=[pl.BlockSpec((3 * BATCH, DIM), lambda: (0, 0))],
        out_specs=pl.BlockSpec((1, DIM), lambda: (0, 0)),
        out_shape=jax.ShapeDtypeStruct((1, DIM), jnp.float32),
    )(gath)


def kernel(picture, text, all_embed, weight, PF_W, PF_b, TF_W, TF_b, w_W,
           edge_index, edge_type, users, pos_items, neg_items):
    head = edge_index[0].astype(jnp.int32)
    tail = edge_index[1].astype(jnp.int32)
    fidx = (edge_type.astype(jnp.int32) - 1) * N_ENT + tail

    item_emb = _attention(picture, text, PF_W, PF_b, TF_W, TF_b, w_W)

    entr0 = _premult(all_embed, weight)
    part1 = _sc_hop(entr0, fidx, head)
    ent1, entr1 = _norm_premult(part1, weight)
    part2 = _sc_hop(entr1, fidx, head)
    res2 = _norm_final(part2, all_embed, ent1)

    gath = _sc_gather(res2, item_emb, users.astype(jnp.int32),
                      pos_items.astype(jnp.int32), neg_items.astype(jnp.int32))
    lossvec = _loss(gath)
    mf_loss = lossvec[0, 0]
    emb_loss = lossvec[0, 1]
    return (mf_loss + emb_loss, mf_loss, emb_loss)


# speedup vs baseline: 15.1296x; 1.0509x over previous
import functools

import jax
import jax.numpy as jnp
from jax import lax
from jax.experimental import pallas as pl
from jax.experimental.pallas import tpu as pltpu
from jax.experimental.pallas import tpu_sc as plsc

N_ENT = 10000
N_ITEMS = 5000
E = 320000
DIM = 128
N_REL = 11
DECAY = 1e-05
BATCH = 1024

NRM1 = N_REL - 1
NC, NS, L = 2, 16, 16
NW = NC * NS
EPW = E // NW
CH = 40
NCHUNK = EPW // CH
ARB = CH
NACH = N_ENT // ARB
GPB = BATCH // NW

IB = 1000
RB = 1000


def _mesh():
    return plsc.VectorSubcoreMesh(core_axis_name="c", subcore_axis_name="s",
                                  num_cores=NC, num_subcores=NS)


def _attn_body(pic_ref, txt_ref, pfw_ref, pfb_ref, tfw_ref, tfb_ref, ww_ref,
               out_ref):
    ox = lax.dot_general(pic_ref[...], pfw_ref[...], (((1,), (1,)), ((), ())),
                         precision=lax.Precision.HIGHEST) + pfb_ref[...]
    oy = lax.dot_general(txt_ref[...], tfw_ref[...], (((1,), (1,)), ((), ())),
                         precision=lax.Precision.HIGHEST) + tfb_ref[...]
    a = jnp.maximum(jnp.sum(ox * ww_ref[...], axis=1, keepdims=True), 0.0)
    b = jnp.maximum(jnp.sum(oy * ww_ref[...], axis=1, keepdims=True), 0.0)
    ea = jnp.exp(a)
    eb = jnp.exp(b)
    out_ref[...] = (ea * ox + eb * oy) / (ea + eb)


def _attention(picture, text, PF_W, PF_b, TF_W, TF_b, w_W):
    return pl.pallas_call(
        _attn_body,
        grid=(N_ITEMS // IB,),
        in_specs=[
            pl.BlockSpec((IB, 2048), lambda i: (i, 0)),
            pl.BlockSpec((IB, 768), lambda i: (i, 0)),
            pl.BlockSpec((DIM, 2048), lambda i: (0, 0)),
            pl.BlockSpec((1, DIM), lambda i: (0, 0)),
            pl.BlockSpec((DIM, 768), lambda i: (0, 0)),
            pl.BlockSpec((1, DIM), lambda i: (0, 0)),
            pl.BlockSpec((1, DIM), lambda i: (0, 0)),
        ],
        out_specs=pl.BlockSpec((IB, DIM), lambda i: (i, 0)),
        out_shape=jax.ShapeDtypeStruct((N_ITEMS, DIM), jnp.float32),
    )(picture, text, PF_W, PF_b.reshape(1, DIM), TF_W, TF_b.reshape(1, DIM),
      w_W)


ER = E // DIM


def _idx_prep_body(ei_ref, et_ref, fidx_ref, head_ref):
    fidx_ref[...] = (et_ref[...] - 1) * N_ENT + ei_ref[1]
    head_ref[...] = ei_ref[0]


def _idx_prep(edge_index, edge_type):
    fidx, head = pl.pallas_call(
        _idx_prep_body,
        in_specs=[
            pl.BlockSpec((2, ER, DIM), lambda: (0, 0, 0)),
            pl.BlockSpec((ER, DIM), lambda: (0, 0)),
        ],
        out_specs=[
            pl.BlockSpec((ER, DIM), lambda: (0, 0)),
            pl.BlockSpec((ER, DIM), lambda: (0, 0)),
        ],
        out_shape=[
            jax.ShapeDtypeStruct((ER, DIM), jnp.int32),
            jax.ShapeDtypeStruct((ER, DIM), jnp.int32),
        ],
    )(edge_index.reshape(2, ER, DIM), edge_type.reshape(ER, DIM))
    return fidx.reshape(E), head.reshape(E)


def _premult_body(ent_ref, w_ref, out_ref):
    for r in range(NRM1):
        out_ref[r] = ent_ref[...] * w_ref[r:r + 1, :]


def _premult(ent, weight):
    out = pl.pallas_call(
        _premult_body,
        grid=(N_ENT // RB,),
        in_specs=[
            pl.BlockSpec((RB, DIM), lambda i: (i, 0)),
            pl.BlockSpec((NRM1, DIM), lambda i: (0, 0)),
        ],
        out_specs=pl.BlockSpec((NRM1, RB, DIM), lambda i: (0, i, 0)),
        out_shape=jax.ShapeDtypeStruct((NRM1, N_ENT, DIM), jnp.float32),
    )(ent, weight)
    return out.reshape(NRM1 * N_ENT, DIM)


def _sum_normalize(part_ref):
    s = part_ref[0] + part_ref[1]
    nrm = jnp.sqrt(jnp.sum(s * s, axis=1, keepdims=True))
    return s / jnp.maximum(nrm, 1e-12)


def _norm_premult_body(part_ref, w_ref, ent_ref, entr_ref):
    ent = _sum_normalize(part_ref)
    ent_ref[...] = ent
    for r in range(NRM1):
        entr_ref[r] = ent * w_ref[r:r + 1, :]


def _norm_premult(part, weight):
    ent, entr = pl.pallas_call(
        _norm_premult_body,
        grid=(N_ENT // RB,),
        in_specs=[
            pl.BlockSpec((NC, RB, DIM), lambda i: (0, i, 0)),
            pl.BlockSpec((NRM1, DIM), lambda i: (0, 0)),
        ],
        out_specs=[
            pl.BlockSpec((RB, DIM), lambda i: (i, 0)),
            pl.BlockSpec((NRM1, RB, DIM), lambda i: (0, i, 0)),
        ],
        out_shape=[
            jax.ShapeDtypeStruct((N_ENT, DIM), jnp.float32),
            jax.ShapeDtypeStruct((NRM1, N_ENT, DIM), jnp.float32),
        ],
    )(part, weight)
    return ent, entr.reshape(NRM1 * N_ENT, DIM)


NB = 5
NROUND = NCHUNK // NB


def _sc_hop(entr, fidx, head):
    @functools.partial(
        pl.kernel,
        out_type=jax.ShapeDtypeStruct((NC * N_ENT, DIM), jnp.float32),
        mesh=_mesh(),
        scratch_types=[
            pltpu.VMEM((EPW,), jnp.int32),
            pltpu.VMEM((EPW,), jnp.int32),
            pltpu.VMEM((NB, CH, DIM), jnp.float32),
            pltpu.VMEM_SHARED((N_ENT, DIM), jnp.float32),
            pltpu.SemaphoreType.DMA((NB,)),
            pltpu.SemaphoreType.DMA((NB,)),
        ],
    )
    def k(entr_hbm, fidx_hbm, head_hbm, out_hbm, fidx_t, head_t, rows_v,
          acc, gsem, ssem):
        c = lax.axis_index("c")
        s = lax.axis_index("s")
        wid = c * NS + s
        zbuf = rows_v.at[0]

        pltpu.sync_copy(fidx_hbm.at[pl.ds(wid * EPW, EPW)], fidx_t)
        pltpu.sync_copy(head_hbm.at[pl.ds(wid * EPW, EPW)], head_t)

        @pl.loop(0, ARB)
        def _(i):
            @pl.loop(0, DIM, step=L)
            def _(j):
                zbuf[i, pl.ds(j, L)] = jnp.zeros((L,), jnp.float32)

        @pl.loop(s, NACH, step=NS)
        def _(g):
            pltpu.sync_copy(zbuf, acc.at[pl.ds(g * ARB, ARB)])

        plsc.subcore_barrier()

        for b in range(NB):
            pltpu.async_copy(entr_hbm.at[fidx_t.at[pl.ds(b * CH, CH)]],
                             rows_v.at[b], gsem.at[b])

        @pl.loop(0, NROUND)
        def _(m):
            for b in range(NB):
                q = m * NB + b
                pltpu.make_async_copy(
                    entr_hbm.at[fidx_t.at[pl.ds(q * CH, CH)]],
                    rows_v.at[b], gsem.at[b]).wait()
                pltpu.async_copy(rows_v.at[b],
                                 acc.at[head_t.at[pl.ds(q * CH, CH)]],
                                 ssem.at[b], add=True)

                @pl.when(m < NROUND - 1)
                def _():
                    pltpu.make_async_copy(
                        rows_v.at[b],
                        acc.at[head_t.at[pl.ds(q * CH, CH)]],
                        ssem.at[b]).wait()
                    pltpu.async_copy(
                        entr_hbm.at[fidx_t.at[pl.ds((q + NB) * CH, CH)]],
                        rows_v.at[b], gsem.at[b])

        for b in range(NB):
            q = NCHUNK - NB + b
            pltpu.make_async_copy(rows_v.at[b],
                                  acc.at[head_t.at[pl.ds(q * CH, CH)]],
                                  ssem.at[b]).wait()

        plsc.subcore_barrier()

        @pl.loop(s, NACH, step=NS)
        def _(g):
            pltpu.sync_copy(acc.at[pl.ds(g * ARB, ARB)],
                            out_hbm.at[pl.ds(c * N_ENT + g * ARB, ARB)])

    return k(entr, fidx, head)


def _sc_gather(all_embed, ent1, part2, item_emb, users, pos_items, neg_items):
    @functools.partial(
        pl.kernel,
        out_type=jax.ShapeDtypeStruct((6 * BATCH, DIM), jnp.float32),
        mesh=_mesh(),
        scratch_types=[
            pltpu.VMEM((GPB,), jnp.int32),
            pltpu.VMEM((GPB,), jnp.int32),
            pltpu.VMEM((GPB, DIM), jnp.float32),
            pltpu.SemaphoreType.DMA,
        ],
    )
    def k(emb_hbm, ent1_hbm, part_hbm, item_hbm, u_hbm, p_hbm, n_hbm,
          out_hbm, idx_v, idx2_v, buf, sem):
        c = lax.axis_index("c")
        s = lax.axis_index("s")
        base = (c * NS + s) * GPB
        pltpu.sync_copy(u_hbm.at[pl.ds(base, GPB)], idx_v)
        pltpu.async_copy(emb_hbm.at[idx_v], buf, sem).wait()
        pltpu.sync_copy(buf, out_hbm.at[pl.ds(base, GPB)])
        pltpu.async_copy(ent1_hbm.at[idx_v], buf, sem).wait()
        pltpu.sync_copy(buf, out_hbm.at[pl.ds(BATCH + base, GPB)])
        pltpu.async_copy(part_hbm.at[idx_v], buf, sem).wait()
        pltpu.sync_copy(buf, out_hbm.at[pl.ds(2 * BATCH + base, GPB)])

        @pl.loop(0, GPB, step=L)
        def _(j):
            idx2_v[pl.ds(j, L)] = idx_v[pl.ds(j, L)] + N_ENT

        pltpu.async_copy(part_hbm.at[idx2_v], buf, sem).wait()
        pltpu.sync_copy(buf, out_hbm.at[pl.ds(3 * BATCH + base, GPB)])
        pltpu.sync_copy(p_hbm.at[pl.ds(base, GPB)], idx_v)
        pltpu.async_copy(item_hbm.at[idx_v], buf, sem).wait()
        pltpu.sync_copy(buf, out_hbm.at[pl.ds(4 * BATCH + base, GPB)])
        pltpu.sync_copy(n_hbm.at[pl.ds(base, GPB)], idx_v)
        pltpu.async_copy(item_hbm.at[idx_v], buf, sem).wait()
        pltpu.sync_copy(buf, out_hbm.at[pl.ds(5 * BATCH + base, GPB)])

    return k(all_embed, ent1, part2, item_emb, users, pos_items, neg_items)


def _loss_body(g_ref, out_ref):
    g = g_ref[...]
    s2 = g[2 * BATCH:3 * BATCH] + g[3 * BATCH:4 * BATCH]
    nrm = jnp.sqrt(jnp.sum(s2 * s2, axis=1, keepdims=True))
    u = g[:BATCH] + g[BATCH:2 * BATCH] + s2 / jnp.maximum(nrm, 1e-12)
    p = g[4 * BATCH:5 * BATCH]
    n = g[5 * BATCH:]
    x = jnp.sum(u * p, axis=1, keepdims=True) - jnp.sum(
        u * n, axis=1, keepdims=True)
    ls = jnp.minimum(x, 0.0) - jnp.log1p(jnp.exp(-jnp.abs(x)))
    mf = -jnp.mean(ls)
    reg = 0.5 * (jnp.sum(u * u) + jnp.sum(p * p) + jnp.sum(n * n))
    emb = jnp.float32(DECAY / BATCH) * reg
    lane = lax.broadcasted_iota(jnp.int32, (1, DIM), 1)
    out_ref[...] = jnp.where(lane == 0, mf, jnp.where(lane == 1, emb, 0.0))


def _loss(gath):
    return pl.pallas_call(
        _loss_body,
        in_specs=[pl.BlockSpec((6 * BATCH, DIM), lambda: (0, 0))],
        out_specs=pl.BlockSpec((1, DIM), lambda: (0, 0)),
        out_shape=jax.ShapeDtypeStruct((1, DIM), jnp.float32),
    )(gath)


def kernel(picture, text, all_embed, weight, PF_W, PF_b, TF_W, TF_b, w_W,
           edge_index, edge_type, users, pos_items, neg_items):
    item_emb = _attention(picture, text, PF_W, PF_b, TF_W, TF_b, w_W)

    fidx, head = _idx_prep(edge_index.astype(jnp.int32),
                           edge_type.astype(jnp.int32))
    entr0 = _premult(all_embed, weight)
    part1 = _sc_hop(entr0, fidx, head)
    ent1, entr1 = _norm_premult(part1.reshape(NC, N_ENT, DIM), weight)
    part2 = _sc_hop(entr1, fidx, head)

    gath = _sc_gather(all_embed, ent1, part2, item_emb,
                      users.astype(jnp.int32),
                      pos_items.astype(jnp.int32), neg_items.astype(jnp.int32))
    lossvec = _loss(gath)
    mf_loss = lossvec[0, 0]
    emb_loss = lossvec[0, 1]
    return (mf_loss + emb_loss, mf_loss, emb_loss)


# speedup vs baseline: 15.5730x; 1.0293x over previous
import functools

import jax
import jax.numpy as jnp
from jax import lax
from jax.experimental import pallas as pl
from jax.experimental.pallas import tpu as pltpu
from jax.experimental.pallas import tpu_sc as plsc

N_ENT = 10000
N_ITEMS = 5000
E = 320000
DIM = 128
N_REL = 11
DECAY = 1e-05
BATCH = 1024

NRM1 = N_REL - 1
NC, NS, L = 2, 16, 16
NW = NC * NS
EPW = E // NW
CH = 40
NCHUNK = EPW // CH
ARB = CH
NACH = N_ENT // ARB
GPB = BATCH // NW

IB = 1000
RB = 1000


def _mesh():
    return plsc.VectorSubcoreMesh(core_axis_name="c", subcore_axis_name="s",
                                  num_cores=NC, num_subcores=NS)


def _attn_body(pic_ref, txt_ref, pfw_ref, pfb_ref, tfw_ref, tfb_ref, ww_ref,
               out_ref):
    ox = lax.dot_general(pic_ref[...], pfw_ref[...], (((1,), (1,)), ((), ())),
                         precision=lax.Precision.HIGHEST) + pfb_ref[...]
    oy = lax.dot_general(txt_ref[...], tfw_ref[...], (((1,), (1,)), ((), ())),
                         precision=lax.Precision.HIGHEST) + tfb_ref[...]
    a = jnp.maximum(jnp.sum(ox * ww_ref[...], axis=1, keepdims=True), 0.0)
    b = jnp.maximum(jnp.sum(oy * ww_ref[...], axis=1, keepdims=True), 0.0)
    ea = jnp.exp(a)
    eb = jnp.exp(b)
    out_ref[...] = (ea * ox + eb * oy) / (ea + eb)


def _attention(picture, text, PF_W, PF_b, TF_W, TF_b, w_W):
    return pl.pallas_call(
        _attn_body,
        grid=(N_ITEMS // IB,),
        in_specs=[
            pl.BlockSpec((IB, 2048), lambda i: (i, 0)),
            pl.BlockSpec((IB, 768), lambda i: (i, 0)),
            pl.BlockSpec((DIM, 2048), lambda i: (0, 0)),
            pl.BlockSpec((1, DIM), lambda i: (0, 0)),
            pl.BlockSpec((DIM, 768), lambda i: (0, 0)),
            pl.BlockSpec((1, DIM), lambda i: (0, 0)),
            pl.BlockSpec((1, DIM), lambda i: (0, 0)),
        ],
        out_specs=pl.BlockSpec((IB, DIM), lambda i: (i, 0)),
        out_shape=jax.ShapeDtypeStruct((N_ITEMS, DIM), jnp.float32),
    )(picture, text, PF_W, PF_b.reshape(1, DIM), TF_W, TF_b.reshape(1, DIM),
      w_W)


ER = E // DIM


def _idx_prep_body(ei_ref, et_ref, fidx_ref, head_ref):
    fidx_ref[...] = (et_ref[...] - 1) * N_ENT + ei_ref[1]
    head_ref[...] = ei_ref[0]


def _idx_prep(edge_index, edge_type):
    return pl.pallas_call(
        _idx_prep_body,
        in_specs=[
            pl.BlockSpec((2, E), lambda: (0, 0)),
            pl.BlockSpec((E,), lambda: (0,)),
        ],
        out_specs=[
            pl.BlockSpec((E,), lambda: (0,)),
            pl.BlockSpec((E,), lambda: (0,)),
        ],
        out_shape=[
            jax.ShapeDtypeStruct((E,), jnp.int32),
            jax.ShapeDtypeStruct((E,), jnp.int32),
        ],
    )(edge_index, edge_type)


def _premult_body(ent_ref, w_ref, out_ref):
    for r in range(NRM1):
        out_ref[r] = ent_ref[...] * w_ref[r:r + 1, :]


def _premult(ent, weight):
    out = pl.pallas_call(
        _premult_body,
        grid=(N_ENT // RB,),
        in_specs=[
            pl.BlockSpec((RB, DIM), lambda i: (i, 0)),
            pl.BlockSpec((NRM1, DIM), lambda i: (0, 0)),
        ],
        out_specs=pl.BlockSpec((NRM1, RB, DIM), lambda i: (0, i, 0)),
        out_shape=jax.ShapeDtypeStruct((NRM1, N_ENT, DIM), jnp.float32),
    )(ent, weight)
    return out.reshape(NRM1 * N_ENT, DIM)


def _sum_normalize(part_ref):
    s = part_ref[0] + part_ref[1]
    nrm = jnp.sqrt(jnp.sum(s * s, axis=1, keepdims=True))
    return s / jnp.maximum(nrm, 1e-12)


def _norm_premult_body(part_ref, w_ref, ent_ref, entr_ref):
    ent = _sum_normalize(part_ref)
    ent_ref[...] = ent
    for r in range(NRM1):
        entr_ref[r] = ent * w_ref[r:r + 1, :]


def _norm_premult(part, weight):
    ent, entr = pl.pallas_call(
        _norm_premult_body,
        grid=(N_ENT // RB,),
        in_specs=[
            pl.BlockSpec((NC, RB, DIM), lambda i: (0, i, 0)),
            pl.BlockSpec((NRM1, DIM), lambda i: (0, 0)),
        ],
        out_specs=[
            pl.BlockSpec((RB, DIM), lambda i: (i, 0)),
            pl.BlockSpec((NRM1, RB, DIM), lambda i: (0, i, 0)),
        ],
        out_shape=[
            jax.ShapeDtypeStruct((N_ENT, DIM), jnp.float32),
            jax.ShapeDtypeStruct((NRM1, N_ENT, DIM), jnp.float32),
        ],
    )(part, weight)
    return ent, entr.reshape(NRM1 * N_ENT, DIM)


NB = 5
NROUND = NCHUNK // NB


def _sc_hop(entr, fidx, head):
    @functools.partial(
        pl.kernel,
        out_type=jax.ShapeDtypeStruct((NC * N_ENT, DIM), jnp.float32),
        mesh=_mesh(),
        scratch_types=[
            pltpu.VMEM((EPW,), jnp.int32),
            pltpu.VMEM((EPW,), jnp.int32),
            pltpu.VMEM((NB, CH, DIM), jnp.float32),
            pltpu.VMEM_SHARED((N_ENT, DIM), jnp.float32),
            pltpu.SemaphoreType.DMA((NB,)),
            pltpu.SemaphoreType.DMA((NB,)),
        ],
    )
    def k(entr_hbm, fidx_hbm, head_hbm, out_hbm, fidx_t, head_t, rows_v,
          acc, gsem, ssem):
        c = lax.axis_index("c")
        s = lax.axis_index("s")
        wid = c * NS + s
        zbuf = rows_v.at[0]

        pltpu.sync_copy(fidx_hbm.at[pl.ds(wid * EPW, EPW)], fidx_t)
        pltpu.sync_copy(head_hbm.at[pl.ds(wid * EPW, EPW)], head_t)

        @pl.loop(0, ARB)
        def _(i):
            @pl.loop(0, DIM, step=L)
            def _(j):
                zbuf[i, pl.ds(j, L)] = jnp.zeros((L,), jnp.float32)

        @pl.loop(s, NACH, step=NS)
        def _(g):
            pltpu.sync_copy(zbuf, acc.at[pl.ds(g * ARB, ARB)])

        plsc.subcore_barrier()

        for b in range(NB):
            pltpu.async_copy(entr_hbm.at[fidx_t.at[pl.ds(b * CH, CH)]],
                             rows_v.at[b], gsem.at[b])

        @pl.loop(0, NROUND)
        def _(m):
            for b in range(NB):
                q = m * NB + b
                pltpu.make_async_copy(
                    entr_hbm.at[fidx_t.at[pl.ds(q * CH, CH)]],
                    rows_v.at[b], gsem.at[b]).wait()
                pltpu.async_copy(rows_v.at[b],
                                 acc.at[head_t.at[pl.ds(q * CH, CH)]],
                                 ssem.at[b], add=True)

                @pl.when(m < NROUND - 1)
                def _():
                    pltpu.make_async_copy(
                        rows_v.at[b],
                        acc.at[head_t.at[pl.ds(q * CH, CH)]],
                        ssem.at[b]).wait()
                    pltpu.async_copy(
                        entr_hbm.at[fidx_t.at[pl.ds((q + NB) * CH, CH)]],
                        rows_v.at[b], gsem.at[b])

        for b in range(NB):
            q = NCHUNK - NB + b
            pltpu.make_async_copy(rows_v.at[b],
                                  acc.at[head_t.at[pl.ds(q * CH, CH)]],
                                  ssem.at[b]).wait()

        plsc.subcore_barrier()

        @pl.loop(s, NACH, step=NS)
        def _(g):
            pltpu.sync_copy(acc.at[pl.ds(g * ARB, ARB)],
                            out_hbm.at[pl.ds(c * N_ENT + g * ARB, ARB)])

    return k(entr, fidx, head)


def _sc_gather(all_embed, ent1, part2, item_emb, users, pos_items, neg_items):
    @functools.partial(
        pl.kernel,
        out_type=jax.ShapeDtypeStruct((6 * BATCH, DIM), jnp.float32),
        mesh=_mesh(),
        scratch_types=[
            pltpu.VMEM((GPB,), jnp.int32),
            pltpu.VMEM((GPB,), jnp.int32),
            pltpu.VMEM((GPB,), jnp.int32),
            pltpu.VMEM((GPB,), jnp.int32),
            pltpu.VMEM((6, GPB, DIM), jnp.float32),
            pltpu.SemaphoreType.DMA((6,)),
            pltpu.SemaphoreType.DMA((6,)),
        ],
    )
    def k(emb_hbm, ent1_hbm, part_hbm, item_hbm, u_hbm, p_hbm, n_hbm,
          out_hbm, uidx_v, idx2_v, pidx_v, nidx_v, bufs, gsem, osem):
        c = lax.axis_index("c")
        s = lax.axis_index("s")
        base = (c * NS + s) * GPB
        pltpu.sync_copy(u_hbm.at[pl.ds(base, GPB)], uidx_v)
        pltpu.sync_copy(p_hbm.at[pl.ds(base, GPB)], pidx_v)
        pltpu.sync_copy(n_hbm.at[pl.ds(base, GPB)], nidx_v)

        @pl.loop(0, GPB, step=L)
        def _(j):
            idx2_v[pl.ds(j, L)] = uidx_v[pl.ds(j, L)] + N_ENT

        srcs = [emb_hbm.at[uidx_v], ent1_hbm.at[uidx_v],
                part_hbm.at[uidx_v], part_hbm.at[idx2_v],
                item_hbm.at[pidx_v], item_hbm.at[nidx_v]]
        for t, src in enumerate(srcs):
            pltpu.async_copy(src, bufs.at[t], gsem.at[t])
        for t, src in enumerate(srcs):
            pltpu.make_async_copy(src, bufs.at[t], gsem.at[t]).wait()
            pltpu.async_copy(bufs.at[t],
                             out_hbm.at[pl.ds(t * BATCH + base, GPB)],
                             osem.at[t])
        for t, src in enumerate(srcs):
            pltpu.make_async_copy(
                bufs.at[t], out_hbm.at[pl.ds(t * BATCH + base, GPB)],
                osem.at[t]).wait()

    return k(all_embed, ent1, part2, item_emb, users, pos_items, neg_items)


def _loss_body(g_ref, out_ref):
    g = g_ref[...]
    s2 = g[2 * BATCH:3 * BATCH] + g[3 * BATCH:4 * BATCH]
    nrm = jnp.sqrt(jnp.sum(s2 * s2, axis=1, keepdims=True))
    u = g[:BATCH] + g[BATCH:2 * BATCH] + s2 / jnp.maximum(nrm, 1e-12)
    p = g[4 * BATCH:5 * BATCH]
    n = g[5 * BATCH:]
    x = jnp.sum(u * p, axis=1, keepdims=True) - jnp.sum(
        u * n, axis=1, keepdims=True)
    ls = jnp.minimum(x, 0.0) - jnp.log1p(jnp.exp(-jnp.abs(x)))
    mf = -jnp.mean(ls)
    reg = 0.5 * (jnp.sum(u * u) + jnp.sum(p * p) + jnp.sum(n * n))
    emb = jnp.float32(DECAY / BATCH) * reg
    lane = lax.broadcasted_iota(jnp.int32, (1, DIM), 1)
    out_ref[...] = jnp.where(lane == 0, mf, jnp.where(lane == 1, emb, 0.0))


def _loss(gath):
    return pl.pallas_call(
        _loss_body,
        in_specs=[pl.BlockSpec((6 * BATCH, DIM), lambda: (0, 0))],
        out_specs=pl.BlockSpec((1, DIM), lambda: (0, 0)),
        out_shape=jax.ShapeDtypeStruct((1, DIM), jnp.float32),
    )(gath)


def kernel(picture, text, all_embed, weight, PF_W, PF_b, TF_W, TF_b, w_W,
           edge_index, edge_type, users, pos_items, neg_items):
    item_emb = _attention(picture, text, PF_W, PF_b, TF_W, TF_b, w_W)

    fidx, head = _idx_prep(edge_index.astype(jnp.int32),
                           edge_type.astype(jnp.int32))
    entr0 = _premult(all_embed, weight)
    part1 = _sc_hop(entr0, fidx, head)
    ent1, entr1 = _norm_premult(part1.reshape(NC, N_ENT, DIM), weight)
    part2 = _sc_hop(entr1, fidx, head)

    gath = _sc_gather(all_embed, ent1, part2, item_emb,
                      users.astype(jnp.int32),
                      pos_items.astype(jnp.int32), neg_items.astype(jnp.int32))
    lossvec = _loss(gath)
    mf_loss = lossvec[0, 0]
    emb_loss = lossvec[0, 1]
    return (mf_loss + emb_loss, mf_loss, emb_loss)


# speedup vs baseline: 15.6348x; 1.0040x over previous
import functools

import jax
import jax.numpy as jnp
from jax import lax
from jax.experimental import pallas as pl
from jax.experimental.pallas import tpu as pltpu
from jax.experimental.pallas import tpu_sc as plsc

N_ENT = 10000
N_ITEMS = 5000
E = 320000
DIM = 128
N_REL = 11
DECAY = 1e-05
BATCH = 1024

NRM1 = N_REL - 1
NC, NS, L = 2, 16, 16
NW = NC * NS
EPW = E // NW
CH = 40
NCHUNK = EPW // CH
ARB = CH
NACH = N_ENT // ARB
GPB = BATCH // NW

IB = 1000
RB = 1000


def _mesh():
    return plsc.VectorSubcoreMesh(core_axis_name="c", subcore_axis_name="s",
                                  num_cores=NC, num_subcores=NS)


def _attn_body(pic_ref, txt_ref, pfw_ref, pfb_ref, tfw_ref, tfb_ref, ww_ref,
               out_ref):
    ox = lax.dot_general(pic_ref[...], pfw_ref[...], (((1,), (1,)), ((), ())),
                         precision=lax.Precision.HIGHEST) + pfb_ref[...]
    oy = lax.dot_general(txt_ref[...], tfw_ref[...], (((1,), (1,)), ((), ())),
                         precision=lax.Precision.HIGHEST) + tfb_ref[...]
    a = jnp.maximum(jnp.sum(ox * ww_ref[...], axis=1, keepdims=True), 0.0)
    b = jnp.maximum(jnp.sum(oy * ww_ref[...], axis=1, keepdims=True), 0.0)
    ea = jnp.exp(a)
    eb = jnp.exp(b)
    out_ref[...] = (ea * ox + eb * oy) / (ea + eb)


def _attention(picture, text, PF_W, PF_b, TF_W, TF_b, w_W):
    return pl.pallas_call(
        _attn_body,
        grid=(N_ITEMS // IB,),
        in_specs=[
            pl.BlockSpec((IB, 2048), lambda i: (i, 0)),
            pl.BlockSpec((IB, 768), lambda i: (i, 0)),
            pl.BlockSpec((DIM, 2048), lambda i: (0, 0)),
            pl.BlockSpec((1, DIM), lambda i: (0, 0)),
            pl.BlockSpec((DIM, 768), lambda i: (0, 0)),
            pl.BlockSpec((1, DIM), lambda i: (0, 0)),
            pl.BlockSpec((1, DIM), lambda i: (0, 0)),
        ],
        out_specs=pl.BlockSpec((IB, DIM), lambda i: (i, 0)),
        out_shape=jax.ShapeDtypeStruct((N_ITEMS, DIM), jnp.float32),
    )(picture, text, PF_W, PF_b.reshape(1, DIM), TF_W, TF_b.reshape(1, DIM),
      w_W)


ER = E // DIM


def _idx_prep_body(ei_ref, et_ref, fidx_ref, head_ref):
    fidx_ref[...] = (et_ref[...] - 1) * N_ENT + ei_ref[1]
    head_ref[...] = ei_ref[0]


def _idx_prep(edge_index, edge_type):
    return pl.pallas_call(
        _idx_prep_body,
        in_specs=[
            pl.BlockSpec((2, E), lambda: (0, 0)),
            pl.BlockSpec((E,), lambda: (0,)),
        ],
        out_specs=[
            pl.BlockSpec((E,), lambda: (0,)),
            pl.BlockSpec((E,), lambda: (0,)),
        ],
        out_shape=[
            jax.ShapeDtypeStruct((E,), jnp.int32),
            jax.ShapeDtypeStruct((E,), jnp.int32),
        ],
    )(edge_index, edge_type)


def _premult_body(ent_ref, w_ref, out_ref):
    for r in range(NRM1):
        out_ref[r] = ent_ref[...] * w_ref[r:r + 1, :]


def _premult(ent, weight):
    out = pl.pallas_call(
        _premult_body,
        grid=(N_ENT // RB,),
        in_specs=[
            pl.BlockSpec((RB, DIM), lambda i: (i, 0)),
            pl.BlockSpec((NRM1, DIM), lambda i: (0, 0)),
        ],
        out_specs=pl.BlockSpec((NRM1, RB, DIM), lambda i: (0, i, 0)),
        out_shape=jax.ShapeDtypeStruct((NRM1, N_ENT, DIM), jnp.float32),
    )(ent, weight)
    return out.reshape(NRM1 * N_ENT, DIM)


def _sum_normalize(part_ref):
    s = part_ref[0] + part_ref[1]
    nrm = jnp.sqrt(jnp.sum(s * s, axis=1, keepdims=True))
    return s / jnp.maximum(nrm, 1e-12)


def _norm_premult_body(part_ref, w_ref, ent_ref, entr_ref):
    ent = _sum_normalize(part_ref)
    ent_ref[...] = ent
    for r in range(NRM1):
        entr_ref[r] = ent * w_ref[r:r + 1, :]


def _norm_premult(part, weight):
    ent, entr = pl.pallas_call(
        _norm_premult_body,
        grid=(N_ENT // RB,),
        in_specs=[
            pl.BlockSpec((NC, RB, DIM), lambda i: (0, i, 0)),
            pl.BlockSpec((NRM1, DIM), lambda i: (0, 0)),
        ],
        out_specs=[
            pl.BlockSpec((RB, DIM), lambda i: (i, 0)),
            pl.BlockSpec((NRM1, RB, DIM), lambda i: (0, i, 0)),
        ],
        out_shape=[
            jax.ShapeDtypeStruct((N_ENT, DIM), jnp.float32),
            jax.ShapeDtypeStruct((NRM1, N_ENT, DIM), jnp.float32),
        ],
    )(part, weight)
    return ent, entr.reshape(NRM1 * N_ENT, DIM)


NB = 6
NROUND = NCHUNK // NB
NEPI = NCHUNK - NROUND * NB


def _sc_hop(entr, fidx, head):
    @functools.partial(
        pl.kernel,
        out_type=jax.ShapeDtypeStruct((NC * N_ENT, DIM), jnp.float32),
        mesh=_mesh(),
        scratch_types=[
            pltpu.VMEM((EPW,), jnp.int32),
            pltpu.VMEM((EPW,), jnp.int32),
            pltpu.VMEM((NB, CH, DIM), jnp.float32),
            pltpu.VMEM_SHARED((N_ENT, DIM), jnp.float32),
            pltpu.SemaphoreType.DMA((NB,)),
            pltpu.SemaphoreType.DMA((NB,)),
        ],
    )
    def k(entr_hbm, fidx_hbm, head_hbm, out_hbm, fidx_t, head_t, rows_v,
          acc, gsem, ssem):
        c = lax.axis_index("c")
        s = lax.axis_index("s")
        wid = c * NS + s
        zbuf = rows_v.at[0]

        pltpu.sync_copy(fidx_hbm.at[pl.ds(wid * EPW, EPW)], fidx_t)
        pltpu.sync_copy(head_hbm.at[pl.ds(wid * EPW, EPW)], head_t)

        @pl.loop(0, ARB)
        def _(i):
            @pl.loop(0, DIM, step=L)
            def _(j):
                zbuf[i, pl.ds(j, L)] = jnp.zeros((L,), jnp.float32)

        @pl.loop(s, NACH, step=NS)
        def _(g):
            pltpu.sync_copy(zbuf, acc.at[pl.ds(g * ARB, ARB)])

        plsc.subcore_barrier()

        for b in range(NB):
            pltpu.async_copy(entr_hbm.at[fidx_t.at[pl.ds(b * CH, CH)]],
                             rows_v.at[b], gsem.at[b])

        @pl.loop(0, NROUND)
        def _(m):
            for b in range(NB):
                q = m * NB + b
                pltpu.make_async_copy(
                    entr_hbm.at[fidx_t.at[pl.ds(q * CH, CH)]],
                    rows_v.at[b], gsem.at[b]).wait()
                pltpu.async_copy(rows_v.at[b],
                                 acc.at[head_t.at[pl.ds(q * CH, CH)]],
                                 ssem.at[b], add=True)

                @pl.when(q + NB < NCHUNK)
                def _():
                    pltpu.make_async_copy(
                        rows_v.at[b],
                        acc.at[head_t.at[pl.ds(q * CH, CH)]],
                        ssem.at[b]).wait()
                    pltpu.async_copy(
                        entr_hbm.at[fidx_t.at[pl.ds((q + NB) * CH, CH)]],
                        rows_v.at[b], gsem.at[b])

        for b in range(NEPI):
            q = NROUND * NB + b
            pltpu.make_async_copy(
                entr_hbm.at[fidx_t.at[pl.ds(q * CH, CH)]],
                rows_v.at[b], gsem.at[b]).wait()
            pltpu.async_copy(rows_v.at[b],
                             acc.at[head_t.at[pl.ds(q * CH, CH)]],
                             ssem.at[b], add=True)
        for b in range(NB):
            q = NROUND * NB + b if b < NEPI else (NROUND - 1) * NB + b
            pltpu.make_async_copy(rows_v.at[b],
                                  acc.at[head_t.at[pl.ds(q * CH, CH)]],
                                  ssem.at[b]).wait()

        plsc.subcore_barrier()

        @pl.loop(s, NACH, step=NS)
        def _(g):
            pltpu.sync_copy(acc.at[pl.ds(g * ARB, ARB)],
                            out_hbm.at[pl.ds(c * N_ENT + g * ARB, ARB)])

    return k(entr, fidx, head)


def _sc_gather(all_embed, ent1, part2, item_emb, users, pos_items, neg_items):
    @functools.partial(
        pl.kernel,
        out_type=jax.ShapeDtypeStruct((6 * BATCH, DIM), jnp.float32),
        mesh=_mesh(),
        scratch_types=[
            pltpu.VMEM((GPB,), jnp.int32),
            pltpu.VMEM((GPB,), jnp.int32),
            pltpu.VMEM((GPB,), jnp.int32),
            pltpu.VMEM((GPB,), jnp.int32),
            pltpu.VMEM((6, GPB, DIM), jnp.float32),
            pltpu.SemaphoreType.DMA((6,)),
            pltpu.SemaphoreType.DMA((6,)),
        ],
    )
    def k(emb_hbm, ent1_hbm, part_hbm, item_hbm, u_hbm, p_hbm, n_hbm,
          out_hbm, uidx_v, idx2_v, pidx_v, nidx_v, bufs, gsem, osem):
        c = lax.axis_index("c")
        s = lax.axis_index("s")
        base = (c * NS + s) * GPB
        pltpu.sync_copy(u_hbm.at[pl.ds(base, GPB)], uidx_v)
        pltpu.sync_copy(p_hbm.at[pl.ds(base, GPB)], pidx_v)
        pltpu.sync_copy(n_hbm.at[pl.ds(base, GPB)], nidx_v)

        @pl.loop(0, GPB, step=L)
        def _(j):
            idx2_v[pl.ds(j, L)] = uidx_v[pl.ds(j, L)] + N_ENT

        srcs = [emb_hbm.at[uidx_v], ent1_hbm.at[uidx_v],
                part_hbm.at[uidx_v], part_hbm.at[idx2_v],
                item_hbm.at[pidx_v], item_hbm.at[nidx_v]]
        for t, src in enumerate(srcs):
            pltpu.async_copy(src, bufs.at[t], gsem.at[t])
        for t, src in enumerate(srcs):
            pltpu.make_async_copy(src, bufs.at[t], gsem.at[t]).wait()
            pltpu.async_copy(bufs.at[t],
                             out_hbm.at[pl.ds(t * BATCH + base, GPB)],
                             osem.at[t])
        for t, src in enumerate(srcs):
            pltpu.make_async_copy(
                bufs.at[t], out_hbm.at[pl.ds(t * BATCH + base, GPB)],
                osem.at[t]).wait()

    return k(all_embed, ent1, part2, item_emb, users, pos_items, neg_items)


def _loss_body(g_ref, out_ref):
    g = g_ref[...]
    s2 = g[2 * BATCH:3 * BATCH] + g[3 * BATCH:4 * BATCH]
    nrm = jnp.sqrt(jnp.sum(s2 * s2, axis=1, keepdims=True))
    u = g[:BATCH] + g[BATCH:2 * BATCH] + s2 / jnp.maximum(nrm, 1e-12)
    p = g[4 * BATCH:5 * BATCH]
    n = g[5 * BATCH:]
    x = jnp.sum(u * p, axis=1, keepdims=True) - jnp.sum(
        u * n, axis=1, keepdims=True)
    ls = jnp.minimum(x, 0.0) - jnp.log1p(jnp.exp(-jnp.abs(x)))
    mf = -jnp.mean(ls)
    reg = 0.5 * (jnp.sum(u * u) + jnp.sum(p * p) + jnp.sum(n * n))
    emb = jnp.float32(DECAY / BATCH) * reg
    lane = lax.broadcasted_iota(jnp.int32, (1, DIM), 1)
    out_ref[...] = jnp.where(lane == 0, mf, jnp.where(lane == 1, emb, 0.0))


def _loss(gath):
    return pl.pallas_call(
        _loss_body,
        in_specs=[pl.BlockSpec((6 * BATCH, DIM), lambda: (0, 0))],
        out_specs=pl.BlockSpec((1, DIM), lambda: (0, 0)),
        out_shape=jax.ShapeDtypeStruct((1, DIM), jnp.float32),
    )(gath)


def kernel(picture, text, all_embed, weight, PF_W, PF_b, TF_W, TF_b, w_W,
           edge_index, edge_type, users, pos_items, neg_items):
    item_emb = _attention(picture, text, PF_W, PF_b, TF_W, TF_b, w_W)

    fidx, head = _idx_prep(edge_index.astype(jnp.int32),
                           edge_type.astype(jnp.int32))
    entr0 = _premult(all_embed, weight)
    part1 = _sc_hop(entr0, fidx, head)
    ent1, entr1 = _norm_premult(part1.reshape(NC, N_ENT, DIM), weight)
    part2 = _sc_hop(entr1, fidx, head)

    gath = _sc_gather(all_embed, ent1, part2, item_emb,
                      users.astype(jnp.int32),
                      pos_items.astype(jnp.int32), neg_items.astype(jnp.int32))
    lossvec = _loss(gath)
    mf_loss = lossvec[0, 0]
    emb_loss = lossvec[0, 1]
    return (mf_loss + emb_loss, mf_loss, emb_loss)


# speedup vs baseline: 16.5245x; 1.0569x over previous
import functools

import jax
import jax.numpy as jnp
from jax import lax
from jax.experimental import pallas as pl
from jax.experimental.pallas import tpu as pltpu
from jax.experimental.pallas import tpu_sc as plsc

N_ENT = 10000
N_ITEMS = 5000
E = 320000
DIM = 128
N_REL = 11
DECAY = 1e-05
BATCH = 1024

NRM1 = N_REL - 1
NC, NS, L = 2, 16, 16
NW = NC * NS
EPW = E // NW
CH = 40
NCHUNK = EPW // CH
ARB = CH
NACH = N_ENT // ARB
GPB = BATCH // NW

IB = 1000
RB = 1000


def _mesh():
    return plsc.VectorSubcoreMesh(core_axis_name="c", subcore_axis_name="s",
                                  num_cores=NC, num_subcores=NS)


def _attn_body(pic_ref, txt_ref, pfw_ref, pfb_ref, tfw_ref, tfb_ref, ww_ref,
               out_ref):
    ox = lax.dot_general(pic_ref[...], pfw_ref[...], (((1,), (1,)), ((), ())),
                         precision=lax.Precision.HIGHEST) + pfb_ref[...]
    oy = lax.dot_general(txt_ref[...], tfw_ref[...], (((1,), (1,)), ((), ())),
                         precision=lax.Precision.HIGHEST) + tfb_ref[...]
    a = jnp.maximum(jnp.sum(ox * ww_ref[...], axis=1, keepdims=True), 0.0)
    b = jnp.maximum(jnp.sum(oy * ww_ref[...], axis=1, keepdims=True), 0.0)
    ea = jnp.exp(a)
    eb = jnp.exp(b)
    out_ref[...] = (ea * ox + eb * oy) / (ea + eb)


def _attention(picture, text, PF_W, PF_b, TF_W, TF_b, w_W):
    return pl.pallas_call(
        _attn_body,
        grid=(N_ITEMS // IB,),
        in_specs=[
            pl.BlockSpec((IB, 2048), lambda i: (i, 0)),
            pl.BlockSpec((IB, 768), lambda i: (i, 0)),
            pl.BlockSpec((DIM, 2048), lambda i: (0, 0)),
            pl.BlockSpec((1, DIM), lambda i: (0, 0)),
            pl.BlockSpec((DIM, 768), lambda i: (0, 0)),
            pl.BlockSpec((1, DIM), lambda i: (0, 0)),
            pl.BlockSpec((1, DIM), lambda i: (0, 0)),
        ],
        out_specs=pl.BlockSpec((IB, DIM), lambda i: (i, 0)),
        out_shape=jax.ShapeDtypeStruct((N_ITEMS, DIM), jnp.float32),
    )(picture, text, PF_W, PF_b.reshape(1, DIM), TF_W, TF_b.reshape(1, DIM),
      w_W)


ER = E // DIM


def _idx_prep_body(ei_ref, et_ref, fidx_ref, head_ref):
    fidx_ref[...] = (et_ref[...] - 1) * N_ENT + ei_ref[1]
    head_ref[...] = ei_ref[0]


def _idx_prep(edge_index, edge_type):
    return pl.pallas_call(
        _idx_prep_body,
        in_specs=[
            pl.BlockSpec((2, E), lambda: (0, 0)),
            pl.BlockSpec((E,), lambda: (0,)),
        ],
        out_specs=[
            pl.BlockSpec((E,), lambda: (0,)),
            pl.BlockSpec((E,), lambda: (0,)),
        ],
        out_shape=[
            jax.ShapeDtypeStruct((E,), jnp.int32),
            jax.ShapeDtypeStruct((E,), jnp.int32),
        ],
    )(edge_index, edge_type)


def _premult_body(ent_ref, w_ref, out_ref):
    for r in range(NRM1):
        out_ref[r] = ent_ref[...] * w_ref[r:r + 1, :]


def _premult(ent, weight):
    out = pl.pallas_call(
        _premult_body,
        grid=(N_ENT // RB,),
        in_specs=[
            pl.BlockSpec((RB, DIM), lambda i: (i, 0)),
            pl.BlockSpec((NRM1, DIM), lambda i: (0, 0)),
        ],
        out_specs=pl.BlockSpec((NRM1, RB, DIM), lambda i: (0, i, 0)),
        out_shape=jax.ShapeDtypeStruct((NRM1, N_ENT, DIM), jnp.float32),
    )(ent, weight)
    return out.reshape(NRM1 * N_ENT, DIM)


def _sum_normalize(part_ref):
    s = part_ref[0] + part_ref[1]
    nrm = jnp.sqrt(jnp.sum(s * s, axis=1, keepdims=True))
    return s / jnp.maximum(nrm, 1e-12)


def _norm_premult_body(part_ref, w_ref, ent_ref, entr_ref):
    ent = _sum_normalize(part_ref)
    ent_ref[...] = ent
    for r in range(NRM1):
        entr_ref[r] = ent * w_ref[r:r + 1, :]


def _norm_premult(part, weight):
    ent, entr = pl.pallas_call(
        _norm_premult_body,
        grid=(N_ENT // RB,),
        in_specs=[
            pl.BlockSpec((NC, RB, DIM), lambda i: (0, i, 0)),
            pl.BlockSpec((NRM1, DIM), lambda i: (0, 0)),
        ],
        out_specs=[
            pl.BlockSpec((RB, DIM), lambda i: (i, 0)),
            pl.BlockSpec((NRM1, RB, DIM), lambda i: (0, i, 0)),
        ],
        out_shape=[
            jax.ShapeDtypeStruct((N_ENT, DIM), jnp.float32),
            jax.ShapeDtypeStruct((NRM1, N_ENT, DIM), jnp.float32),
        ],
    )(part, weight)
    return ent, entr.reshape(NRM1 * N_ENT, DIM)


NB = 6
NROUND = NCHUNK // NB
NEPI = NCHUNK - NROUND * NB


def _sc_hop(entr, fidx, head):
    @functools.partial(
        pl.kernel,
        out_type=jax.ShapeDtypeStruct((NC * N_ENT, DIM), jnp.float32),
        mesh=_mesh(),
        scratch_types=[
            pltpu.VMEM((EPW,), jnp.int32),
            pltpu.VMEM((EPW,), jnp.int32),
            pltpu.VMEM((NB, CH, DIM), jnp.float32),
            pltpu.VMEM_SHARED((N_ENT, DIM), jnp.float32),
            pltpu.SemaphoreType.DMA((NB,)),
            pltpu.SemaphoreType.DMA((NB,)),
        ],
    )
    def k(entr_hbm, fidx_hbm, head_hbm, out_hbm, fidx_t, head_t, rows_v,
          acc, gsem, ssem):
        c = lax.axis_index("c")
        s = lax.axis_index("s")
        wid = c * NS + s
        zbuf = rows_v.at[0]

        @pl.loop(0, ARB)
        def _(i):
            @pl.loop(0, DIM, step=L)
            def _(j):
                zbuf[i, pl.ds(j, L)] = jnp.zeros((L,), jnp.float32)

        pltpu.async_copy(fidx_hbm.at[pl.ds(wid * EPW, EPW)], fidx_t,
                         gsem.at[0])
        pltpu.async_copy(head_hbm.at[pl.ds(wid * EPW, EPW)], head_t,
                         gsem.at[1])

        @pl.loop(s, NACH, step=NS)
        def _(g):
            pltpu.async_copy(zbuf, acc.at[pl.ds(g * ARB, ARB)], ssem.at[0])

        @pl.loop(s, NACH, step=NS)
        def _(g):
            pltpu.make_async_copy(zbuf, acc.at[pl.ds(g * ARB, ARB)],
                                  ssem.at[0]).wait()

        pltpu.make_async_copy(fidx_hbm.at[pl.ds(wid * EPW, EPW)], fidx_t,
                              gsem.at[0]).wait()
        pltpu.make_async_copy(head_hbm.at[pl.ds(wid * EPW, EPW)], head_t,
                              gsem.at[1]).wait()
        plsc.subcore_barrier()

        for b in range(NB):
            pltpu.async_copy(entr_hbm.at[fidx_t.at[pl.ds(b * CH, CH)]],
                             rows_v.at[b], gsem.at[b])

        @pl.loop(0, NROUND)
        def _(m):
            for b in range(NB):
                q = m * NB + b
                pltpu.make_async_copy(
                    entr_hbm.at[fidx_t.at[pl.ds(q * CH, CH)]],
                    rows_v.at[b], gsem.at[b]).wait()
                pltpu.async_copy(rows_v.at[b],
                                 acc.at[head_t.at[pl.ds(q * CH, CH)]],
                                 ssem.at[b], add=True)

                @pl.when(q + NB < NCHUNK)
                def _():
                    pltpu.make_async_copy(
                        rows_v.at[b],
                        acc.at[head_t.at[pl.ds(q * CH, CH)]],
                        ssem.at[b]).wait()
                    pltpu.async_copy(
                        entr_hbm.at[fidx_t.at[pl.ds((q + NB) * CH, CH)]],
                        rows_v.at[b], gsem.at[b])

        for b in range(NEPI):
            q = NROUND * NB + b
            pltpu.make_async_copy(
                entr_hbm.at[fidx_t.at[pl.ds(q * CH, CH)]],
                rows_v.at[b], gsem.at[b]).wait()
            pltpu.async_copy(rows_v.at[b],
                             acc.at[head_t.at[pl.ds(q * CH, CH)]],
                             ssem.at[b], add=True)
        for b in range(NB):
            q = NROUND * NB + b if b < NEPI else (NROUND - 1) * NB + b
            pltpu.make_async_copy(rows_v.at[b],
                                  acc.at[head_t.at[pl.ds(q * CH, CH)]],
                                  ssem.at[b]).wait()

        plsc.subcore_barrier()

        @pl.loop(s, NACH, step=NS)
        def _(g):
            pltpu.async_copy(acc.at[pl.ds(g * ARB, ARB)],
                             out_hbm.at[pl.ds(c * N_ENT + g * ARB, ARB)],
                             ssem.at[0])

        @pl.loop(s, NACH, step=NS)
        def _(g):
            pltpu.make_async_copy(
                acc.at[pl.ds(g * ARB, ARB)],
                out_hbm.at[pl.ds(c * N_ENT + g * ARB, ARB)],
                ssem.at[0]).wait()

    return k(entr, fidx, head)


def _sc_gather(all_embed, ent1, part2, item_emb, users, pos_items, neg_items):
    @functools.partial(
        pl.kernel,
        out_type=jax.ShapeDtypeStruct((6 * BATCH, DIM), jnp.float32),
        mesh=_mesh(),
        scratch_types=[
            pltpu.VMEM((GPB,), jnp.int32),
            pltpu.VMEM((GPB,), jnp.int32),
            pltpu.VMEM((GPB,), jnp.int32),
            pltpu.VMEM((GPB,), jnp.int32),
            pltpu.VMEM((6, GPB, DIM), jnp.float32),
            pltpu.SemaphoreType.DMA((6,)),
            pltpu.SemaphoreType.DMA((6,)),
        ],
    )
    def k(emb_hbm, ent1_hbm, part_hbm, item_hbm, u_hbm, p_hbm, n_hbm,
          out_hbm, uidx_v, idx2_v, pidx_v, nidx_v, bufs, gsem, osem):
        c = lax.axis_index("c")
        s = lax.axis_index("s")
        base = (c * NS + s) * GPB
        pltpu.sync_copy(u_hbm.at[pl.ds(base, GPB)], uidx_v)
        pltpu.sync_copy(p_hbm.at[pl.ds(base, GPB)], pidx_v)
        pltpu.sync_copy(n_hbm.at[pl.ds(base, GPB)], nidx_v)

        @pl.loop(0, GPB, step=L)
        def _(j):
            idx2_v[pl.ds(j, L)] = uidx_v[pl.ds(j, L)] + N_ENT

        srcs = [emb_hbm.at[uidx_v], ent1_hbm.at[uidx_v],
                part_hbm.at[uidx_v], part_hbm.at[idx2_v],
                item_hbm.at[pidx_v], item_hbm.at[nidx_v]]
        for t, src in enumerate(srcs):
            pltpu.async_copy(src, bufs.at[t], gsem.at[t])
        for t, src in enumerate(srcs):
            pltpu.make_async_copy(src, bufs.at[t], gsem.at[t]).wait()
            pltpu.async_copy(bufs.at[t],
                             out_hbm.at[pl.ds(t * BATCH + base, GPB)],
                             osem.at[t])
        for t, src in enumerate(srcs):
            pltpu.make_async_copy(
                bufs.at[t], out_hbm.at[pl.ds(t * BATCH + base, GPB)],
                osem.at[t]).wait()

    return k(all_embed, ent1, part2, item_emb, users, pos_items, neg_items)


def _loss_body(g_ref, out_ref):
    g = g_ref[...]
    s2 = g[2 * BATCH:3 * BATCH] + g[3 * BATCH:4 * BATCH]
    nrm = jnp.sqrt(jnp.sum(s2 * s2, axis=1, keepdims=True))
    u = g[:BATCH] + g[BATCH:2 * BATCH] + s2 / jnp.maximum(nrm, 1e-12)
    p = g[4 * BATCH:5 * BATCH]
    n = g[5 * BATCH:]
    x = jnp.sum(u * p, axis=1, keepdims=True) - jnp.sum(
        u * n, axis=1, keepdims=True)
    ls = jnp.minimum(x, 0.0) - jnp.log1p(jnp.exp(-jnp.abs(x)))
    mf = -jnp.mean(ls)
    reg = 0.5 * (jnp.sum(u * u) + jnp.sum(p * p) + jnp.sum(n * n))
    emb = jnp.float32(DECAY / BATCH) * reg
    lane = lax.broadcasted_iota(jnp.int32, (1, DIM), 1)
    out_ref[...] = jnp.where(lane == 0, mf, jnp.where(lane == 1, emb, 0.0))


def _loss(gath):
    return pl.pallas_call(
        _loss_body,
        in_specs=[pl.BlockSpec((6 * BATCH, DIM), lambda: (0, 0))],
        out_specs=pl.BlockSpec((1, DIM), lambda: (0, 0)),
        out_shape=jax.ShapeDtypeStruct((1, DIM), jnp.float32),
    )(gath)


def kernel(picture, text, all_embed, weight, PF_W, PF_b, TF_W, TF_b, w_W,
           edge_index, edge_type, users, pos_items, neg_items):
    item_emb = _attention(picture, text, PF_W, PF_b, TF_W, TF_b, w_W)

    fidx, head = _idx_prep(edge_index.astype(jnp.int32),
                           edge_type.astype(jnp.int32))
    entr0 = _premult(all_embed, weight)
    part1 = _sc_hop(entr0, fidx, head)
    ent1, entr1 = _norm_premult(part1.reshape(NC, N_ENT, DIM), weight)
    part2 = _sc_hop(entr1, fidx, head)

    gath = _sc_gather(all_embed, ent1, part2, item_emb,
                      users.astype(jnp.int32),
                      pos_items.astype(jnp.int32), neg_items.astype(jnp.int32))
    lossvec = _loss(gath)
    mf_loss = lossvec[0, 0]
    emb_loss = lossvec[0, 1]
    return (mf_loss + emb_loss, mf_loss, emb_loss)
